# Initial kernel scaffold; baseline (speedup 1.0000x reference)
import jax
import jax.numpy as jnp
from jax.experimental import pallas as pl


def kernel(H_0, X_0, cond_embedding, chain_ids, generate_mask, lengths, t, Win1, bin1, Win2, bin2, Win3, bin3, Wh2i, bh2i, Eemb, We_a, be_a, We_b, be_b, Wx, bx, Wn_a, bn_a, Wn_b, bn_b):
    raise NotImplementedError("write your pallas kernel here")


# speedup vs baseline: 6.9729x; 6.9729x over previous
import numpy as np
import jax
import jax.numpy as jnp
from jax.experimental import pallas as pl
from jax.experimental.pallas import tpu as pltpu

LATENT = 16
HIDDEN = 128
NSTEPS = 100
NLAYERS = 3
B = 64
NPAD = 64
XPAD = 8

_offs = (np.arange(B) * (np.arange(B) - 1)) // 2
_r = np.arange(NPAD)
_SRC_NP = (_offs[:, None]
           + np.minimum(_r[None, :], np.maximum(np.arange(B)[:, None] - 1, 0))
           ).astype(np.int32)
_VALID_NP = (_r[None, :] < np.arange(B)[:, None])
_FREQS_NP = np.exp(-np.log(10000.0) * np.arange(HIDDEN // 2)
                   / (HIDDEN // 2 - 1)).astype(np.float32).reshape(1, -1)


def _body(ab_s, beta_s, bx_s,
          H0b, X0b, Cb, eXb, eHb, cidb, gmb,
          freqs_in, Win1, bin1, Win2, bin2, Win3, bin3,
          A1, A2, wd3, Ce0, Ce1, bea, Web, beb, WxT,
          Wna, bna, Wnb, bnb, Wh2i, bh2i,
          out):
    b = pl.program_id(0)

    @pl.when(b == 0)
    def _init():
        out[0] = 0.0
        out[1] = 0.0
        out[2] = 0.0

    ab = ab_s[b]
    beta = beta_s[b]
    sa = jnp.sqrt(ab)
    sb = jnp.sqrt(1.0 - ab)
    gm = gmb[0, 0, :].reshape(NPAD, 1)
    H0 = H0b[0]
    X0 = X0b[0]
    cond = Cb[0]
    eX = eXb[0]
    eH = eHb[0]

    Xn0 = gm * (sa * X0 + sb * eX) + (1.0 - gm) * X0
    Hn = gm * (sa * H0 + sb * eH) + (1.0 - gm) * H0

    freqs = freqs_in[...]
    args = beta * freqs
    te = jnp.concatenate([jnp.sin(args), jnp.cos(args)], axis=1)
    te = jnp.broadcast_to(te, (NPAD, HIDDEN))

    feat = jnp.concatenate([Hn, cond, te], axis=1)
    h = jnp.maximum(
        jnp.dot(feat, Win1[...], preferred_element_type=jnp.float32)
        + bin1[...], 0.0)
    h = jnp.maximum(
        jnp.dot(h, Win2[...], preferred_element_type=jnp.float32)
        + bin2[...], 0.0)
    h = jnp.dot(h, Win3[...], preferred_element_type=jnp.float32) + bin3[...]

    cid = cidb[0]
    CEm = (cid.reshape(NPAD, 1) == cid.reshape(1, NPAD)).astype(jnp.float32)
    jota = jax.lax.broadcasted_iota(jnp.int32, (1, NPAD), 1)
    maskj = (jota < b).astype(jnp.float32)
    Lf = b.astype(jnp.float32)
    inv = 1.0 / (Lf + 1.0)

    Xn = Xn0
    for l in range(NLAYERS):
        P = (jnp.dot(h, A1[l], preferred_element_type=jnp.float32)
             + bea[l])
        Q = jnp.dot(h, A2[l], preferred_element_type=jnp.float32)
        n2 = jnp.sum(Xn * Xn, axis=1, keepdims=True)
        G = jnp.dot(Xn, Xn.T, preferred_element_type=jnp.float32)
        D2 = n2 + n2.reshape(1, NPAD) - 2.0 * G
        ce0 = Ce0[l]
        ce1 = Ce1[l]
        E = (P.reshape(NPAD, 1, HIDDEN)
             + Q.reshape(1, NPAD, HIDDEN)
             + D2.reshape(NPAD, NPAD, 1) * wd3[l].reshape(1, 1, HIDDEN)
             + (ce1.reshape(1, 1, HIDDEN)
                + CEm.reshape(NPAD, NPAD, 1)
                * (ce0 - ce1).reshape(1, 1, HIDDEN)))
        M1 = jnp.maximum(E, 0.0).reshape(NPAD * NPAD, HIDDEN)
        M2 = jnp.maximum(
            jnp.dot(M1, Web[l], preferred_element_type=jnp.float32)
            + beb[l], 0.0)
        xw = jnp.tanh(
            jnp.sum(M2 * WxT[l], axis=1, keepdims=True) + bx_s[l])
        M2m = M2.reshape(NPAD, NPAD, HIDDEN) * maskj.reshape(1, NPAD, 1)
        XW = xw.reshape(NPAD, NPAD) * maskj
        agg = jnp.sum(M2m, axis=1) * inv
        S = jnp.sum(XW, axis=1, keepdims=True)
        Xn = Xn + (Xn * S
                   - jnp.dot(XW, Xn, preferred_element_type=jnp.float32)) * inv
        nh = jnp.concatenate([h, agg], axis=1)
        h = (h
             + jnp.dot(
                 jnp.maximum(
                     jnp.dot(nh, Wna[l], preferred_element_type=jnp.float32)
                     + bna[l], 0.0),
                 Wnb[l], preferred_element_type=jnp.float32)
             + bnb[l])

    nH = jnp.dot(h, Wh2i[...], preferred_element_type=jnp.float32) + bh2i[...]
    exd = (Xn - Xn0) - eX
    pX = jnp.sum(jnp.sum(exd * exd, axis=1, keepdims=True) * gm)
    ehd = (nH - Hn) - eH
    pH = jnp.sum(jnp.sum(ehd * ehd, axis=1, keepdims=True) * gm)
    pD = jnp.sum(gm)
    out[0] += pX
    out[1] += pH
    out[2] += pD


def kernel(H_0, X_0, cond_embedding, chain_ids, generate_mask, lengths, t,
           Win1, bin1, Win2, bin2, Win3, bin3, Wh2i, bh2i, Eemb,
           We_a, be_a, We_b, be_b, Wx, bx, Wn_a, bn_a, Wn_b, bn_b):
    kx, kh = jax.random.split(jax.random.key(42))
    eps_X = jax.random.normal(kx, X_0.shape, dtype=jnp.float32)
    eps_H = jax.random.normal(kh, H_0.shape, dtype=jnp.float32)
    betas = jnp.concatenate([jnp.zeros(1), jnp.linspace(1e-4, 0.02, NSTEPS)])
    abars = jnp.cumprod(1.0 - betas)
    ab_vec = abars[t].astype(jnp.float32)
    beta_vec = betas[t].astype(jnp.float32)

    src = jnp.asarray(_SRC_NP).reshape(-1)
    vf = jnp.asarray(_VALID_NP.astype(np.float32))

    def padc(a, c):
        ap = a[src].reshape(B, NPAD, a.shape[1]) * vf[:, :, None]
        if a.shape[1] < c:
            ap = jnp.pad(ap, ((0, 0), (0, 0), (0, c - a.shape[1])))
        return ap

    H0p = padc(H_0, LATENT)
    X0p = padc(X_0, XPAD)
    condp = padc(cond_embedding, HIDDEN)
    eXp = padc(eps_X, XPAD)
    eHp = padc(eps_H, LATENT)
    cidp = chain_ids.astype(jnp.int32)[src].reshape(B, 1, NPAD)
    gmp = (generate_mask.astype(jnp.float32)[src].reshape(B, 1, NPAD)
           * vf.reshape(B, 1, NPAD))

    A1 = We_a[:, :HIDDEN, :]
    A2 = We_a[:, HIDDEN:2 * HIDDEN, :]
    wd3 = We_a[:, 2 * HIDDEN:2 * HIDDEN + 1, :]
    CeW = jnp.einsum('ec,lch->leh', Eemb, We_a[:, 2 * HIDDEN + 1:, :])
    Ce0 = CeW[:, 0:1, :].reshape(NLAYERS, 1, HIDDEN)
    Ce1 = CeW[:, 1:2, :].reshape(NLAYERS, 1, HIDDEN)

    blk = lambda shp: pl.BlockSpec(shp, lambda b: (b, 0, 0))
    rep = lambda arr: pl.BlockSpec(arr.shape,
                                   lambda b, _n=arr.ndim: (0,) * _n)
    smem = pl.BlockSpec(memory_space=pltpu.SMEM)

    weights = [jnp.asarray(_FREQS_NP),
               Win1, bin1.reshape(1, HIDDEN), Win2, bin2.reshape(1, HIDDEN),
               Win3, bin3.reshape(1, HIDDEN),
               A1, A2, wd3, Ce0, Ce1, be_a.reshape(NLAYERS, 1, HIDDEN),
               We_b, be_b.reshape(NLAYERS, 1, HIDDEN),
               Wx.reshape(NLAYERS, 1, HIDDEN),
               Wn_a, bn_a.reshape(NLAYERS, 1, HIDDEN),
               Wn_b, bn_b.reshape(NLAYERS, 1, HIDDEN),
               Wh2i, bh2i.reshape(1, LATENT)]

    res = pl.pallas_call(
        _body,
        grid=(B,),
        in_specs=[smem, smem, smem,
                  blk((1, NPAD, LATENT)), blk((1, NPAD, XPAD)),
                  blk((1, NPAD, HIDDEN)), blk((1, NPAD, XPAD)),
                  blk((1, NPAD, LATENT)),
                  blk((1, 1, NPAD)), blk((1, 1, NPAD))]
                 + [rep(w) for w in weights],
        out_specs=pl.BlockSpec(memory_space=pltpu.SMEM),
        out_shape=jax.ShapeDtypeStruct((3,), jnp.float32),
        compiler_params=pltpu.CompilerParams(
            dimension_semantics=("arbitrary",)),
    )(ab_vec, beta_vec, bx.reshape(-1),
      H0p, X0p, condp, eXp, eHp, cidp, gmp, *weights)

    denom = res[2] + 1e-8
    return jnp.stack([res[0] / denom, res[1] / denom])


# speedup vs baseline: 12.7304x; 1.8257x over previous
import numpy as np
import jax
import jax.numpy as jnp
from jax.experimental import pallas as pl
from jax.experimental.pallas import tpu as pltpu

LATENT = 16
HIDDEN = 128
NSTEPS = 100
NLAYERS = 3
B = 64
NPAD = 64
XPAD = 8

NT = B // 2
_offs = (np.arange(B) * (np.arange(B) - 1)) // 2
_r = np.arange(NPAD)[None, :]
_p = np.arange(NT)[:, None]
_in1 = _r < _p
_in2 = (_r >= _p) & (_r < NPAD - 1)
_SRC_NP = np.where(
    _in1, _offs[np.minimum(_p, B - 1)] + _r,
    np.where(_in2, _offs[B - 1 - _p] + (_r - _p), 0)).astype(np.int32)
_VALID_NP = (_in1 | _in2)
_FREQS_NP = np.exp(-np.log(10000.0) * np.arange(HIDDEN // 2)
                   / (HIDDEN // 2 - 1)).astype(np.float32).reshape(1, -1)


def _body(ab_s, beta_s, bx_s,
          H0b, X0b, Cb, eXb, eHb, cidb, gmb,
          freqs_in, Win1, bin1, Win2, bin2, Win3, bin3,
          A1, A2, wd3, Ce0, Ce1, bea, Web, beb, WxT,
          Wna, bna, Wnb, bnb, Wh2i, bh2i,
          out):
    p = pl.program_id(0)

    @pl.when(p == 0)
    def _init():
        out[0] = 0.0
        out[1] = 0.0
        out[2] = 0.0

    iota_r = jax.lax.broadcasted_iota(jnp.int32, (NPAD, 1), 0)
    rowsel = (iota_r < p).astype(jnp.float32)
    ab1 = ab_s[p]
    ab2 = ab_s[B - 1 - p]
    abr = rowsel * ab1 + (1.0 - rowsel) * ab2
    sa = jnp.sqrt(abr)
    sb = jnp.sqrt(1.0 - abr)
    gm = gmb[0, 0, :].reshape(NPAD, 1)
    H0 = H0b[0]
    X0 = X0b[0]
    cond = Cb[0]
    eX = eXb[0]
    eH = eHb[0]

    Xn0 = gm * (sa * X0 + sb * eX) + (1.0 - gm) * X0
    Hn = gm * (sa * H0 + sb * eH) + (1.0 - gm) * H0

    freqs = freqs_in[...]
    te1 = beta_s[p] * freqs
    te2 = beta_s[B - 1 - p] * freqs
    te1 = jnp.concatenate([jnp.sin(te1), jnp.cos(te1)], axis=1)
    te2 = jnp.concatenate([jnp.sin(te2), jnp.cos(te2)], axis=1)
    te = rowsel * te1 + (1.0 - rowsel) * te2

    feat = jnp.concatenate([Hn, cond, te], axis=1)
    h = jnp.maximum(
        jnp.dot(feat, Win1[...], preferred_element_type=jnp.float32)
        + bin1[...], 0.0)
    h = jnp.maximum(
        jnp.dot(h, Win2[...], preferred_element_type=jnp.float32)
        + bin2[...], 0.0)
    h = jnp.dot(h, Win3[...], preferred_element_type=jnp.float32) + bin3[...]

    cid = cidb[0]
    CEm = (cid.reshape(NPAD, 1) == cid.reshape(1, NPAD)).astype(jnp.float32)
    jota = jax.lax.broadcasted_iota(jnp.int32, (1, NPAD), 1)
    validj = (jota < NPAD - 1).astype(jnp.float32)
    rsj = rowsel.reshape(1, NPAD)
    Mpair = (rowsel * rsj + (1.0 - rowsel) * (1.0 - rsj)) * validj
    pf = p.astype(jnp.float32)
    invr = rowsel / (pf + 1.0) + (1.0 - rowsel) / (64.0 - pf)

    Xn = Xn0
    for l in range(NLAYERS):
        P = (jnp.dot(h, A1[l], preferred_element_type=jnp.float32)
             + bea[l])
        Q = jnp.dot(h, A2[l], preferred_element_type=jnp.float32)
        n2 = jnp.sum(Xn * Xn, axis=1, keepdims=True)
        G = jnp.dot(Xn, Xn.T, preferred_element_type=jnp.float32)
        D2 = n2 + n2.reshape(1, NPAD) - 2.0 * G
        ce0 = Ce0[l]
        ce1 = Ce1[l]
        E = (P.reshape(NPAD, 1, HIDDEN)
             + Q.reshape(1, NPAD, HIDDEN)
             + D2.reshape(NPAD, NPAD, 1) * wd3[l].reshape(1, 1, HIDDEN)
             + (ce1.reshape(1, 1, HIDDEN)
                + CEm.reshape(NPAD, NPAD, 1)
                * (ce0 - ce1).reshape(1, 1, HIDDEN)))
        M1 = jnp.maximum(E, 0.0).reshape(NPAD * NPAD, HIDDEN)
        M2 = jnp.maximum(
            jnp.dot(M1, Web[l], preferred_element_type=jnp.float32)
            + beb[l], 0.0)
        xw = jnp.tanh(
            jnp.sum(M2 * WxT[l], axis=1, keepdims=True) + bx_s[l])
        M2m = M2.reshape(NPAD, NPAD, HIDDEN) * Mpair.reshape(NPAD, NPAD, 1)
        XW = xw.reshape(NPAD, NPAD) * Mpair
        agg = jnp.sum(M2m, axis=1) * invr
        S = jnp.sum(XW, axis=1, keepdims=True)
        Xn = Xn + (Xn * S
                   - jnp.dot(XW, Xn,
                             preferred_element_type=jnp.float32)) * invr
        nh = jnp.concatenate([h, agg], axis=1)
        h = (h
             + jnp.dot(
                 jnp.maximum(
                     jnp.dot(nh, Wna[l], preferred_element_type=jnp.float32)
                     + bna[l], 0.0),
                 Wnb[l], preferred_element_type=jnp.float32)
             + bnb[l])

    nH = jnp.dot(h, Wh2i[...], preferred_element_type=jnp.float32) + bh2i[...]
    exd = (Xn - Xn0) - eX
    pX = jnp.sum(jnp.sum(exd * exd, axis=1, keepdims=True) * gm)
    ehd = (nH - Hn) - eH
    pH = jnp.sum(jnp.sum(ehd * ehd, axis=1, keepdims=True) * gm)
    pD = jnp.sum(gm)
    out[0] += pX
    out[1] += pH
    out[2] += pD


def kernel(H_0, X_0, cond_embedding, chain_ids, generate_mask, lengths, t,
           Win1, bin1, Win2, bin2, Win3, bin3, Wh2i, bh2i, Eemb,
           We_a, be_a, We_b, be_b, Wx, bx, Wn_a, bn_a, Wn_b, bn_b):
    kx, kh = jax.random.split(jax.random.key(42))
    eps_X = jax.random.normal(kx, X_0.shape, dtype=jnp.float32)
    eps_H = jax.random.normal(kh, H_0.shape, dtype=jnp.float32)
    betas = jnp.concatenate([jnp.zeros(1), jnp.linspace(1e-4, 0.02, NSTEPS)])
    abars = jnp.cumprod(1.0 - betas)
    ab_vec = abars[t].astype(jnp.float32)
    beta_vec = betas[t].astype(jnp.float32)

    src = jnp.asarray(_SRC_NP).reshape(-1)
    vf = jnp.asarray(_VALID_NP.astype(np.float32))

    def padc(a, c):
        ap = a[src].reshape(NT, NPAD, a.shape[1]) * vf[:, :, None]
        if a.shape[1] < c:
            ap = jnp.pad(ap, ((0, 0), (0, 0), (0, c - a.shape[1])))
        return ap

    H0p = padc(H_0, LATENT)
    X0p = padc(X_0, XPAD)
    condp = padc(cond_embedding, HIDDEN)
    eXp = padc(eps_X, XPAD)
    eHp = padc(eps_H, LATENT)
    cidp = chain_ids.astype(jnp.int32)[src].reshape(NT, 1, NPAD)
    gmp = (generate_mask.astype(jnp.float32)[src].reshape(NT, 1, NPAD)
           * vf.reshape(NT, 1, NPAD))

    A1 = We_a[:, :HIDDEN, :]
    A2 = We_a[:, HIDDEN:2 * HIDDEN, :]
    wd3 = We_a[:, 2 * HIDDEN:2 * HIDDEN + 1, :]
    CeW = jnp.einsum('ec,lch->leh', Eemb, We_a[:, 2 * HIDDEN + 1:, :])
    Ce0 = CeW[:, 0:1, :].reshape(NLAYERS, 1, HIDDEN)
    Ce1 = CeW[:, 1:2, :].reshape(NLAYERS, 1, HIDDEN)

    blk = lambda shp: pl.BlockSpec(shp, lambda b: (b, 0, 0))
    rep = lambda arr: pl.BlockSpec(arr.shape,
                                   lambda b, _n=arr.ndim: (0,) * _n)
    smem = pl.BlockSpec(memory_space=pltpu.SMEM)

    weights = [jnp.asarray(_FREQS_NP),
               Win1, bin1.reshape(1, HIDDEN), Win2, bin2.reshape(1, HIDDEN),
               Win3, bin3.reshape(1, HIDDEN),
               A1, A2, wd3, Ce0, Ce1, be_a.reshape(NLAYERS, 1, HIDDEN),
               We_b, be_b.reshape(NLAYERS, 1, HIDDEN),
               Wx.reshape(NLAYERS, 1, HIDDEN),
               Wn_a, bn_a.reshape(NLAYERS, 1, HIDDEN),
               Wn_b, bn_b.reshape(NLAYERS, 1, HIDDEN),
               Wh2i, bh2i.reshape(1, LATENT)]

    res = pl.pallas_call(
        _body,
        grid=(NT,),
        in_specs=[smem, smem, smem,
                  blk((1, NPAD, LATENT)), blk((1, NPAD, XPAD)),
                  blk((1, NPAD, HIDDEN)), blk((1, NPAD, XPAD)),
                  blk((1, NPAD, LATENT)),
                  blk((1, 1, NPAD)), blk((1, 1, NPAD))]
                 + [rep(w) for w in weights],
        out_specs=pl.BlockSpec(memory_space=pltpu.SMEM),
        out_shape=jax.ShapeDtypeStruct((3,), jnp.float32),
        compiler_params=pltpu.CompilerParams(
            dimension_semantics=("arbitrary",)),
    )(ab_vec, beta_vec, bx.reshape(-1),
      H0p, X0p, condp, eXp, eHp, cidp, gmp, *weights)

    denom = res[2] + 1e-8
    return jnp.stack([res[0] / denom, res[1] / denom])


# speedup vs baseline: 14.9536x; 1.1746x over previous
import numpy as np
import jax
import jax.numpy as jnp
from jax.experimental import pallas as pl
from jax.experimental.pallas import tpu as pltpu

LATENT = 16
HIDDEN = 128
NSTEPS = 100
NLAYERS = 3
B = 64
NPAD = 64
XPAD = 8

NT = B // 2
_offs = (np.arange(B) * (np.arange(B) - 1)) // 2
_r = np.arange(NPAD)[None, :]
_p = np.arange(NT)[:, None]
_in1 = _r < _p
_in2 = (_r >= _p) & (_r < NPAD - 1)
_SRC_NP = np.where(
    _in1, _offs[np.minimum(_p, B - 1)] + _r,
    np.where(_in2, _offs[B - 1 - _p] + (_r - _p), 0)).astype(np.int32)
_VALID_NP = (_in1 | _in2)
_FREQS_NP = np.exp(-np.log(10000.0) * np.arange(HIDDEN // 2)
                   / (HIDDEN // 2 - 1)).astype(np.float32).reshape(1, -1)


def _body(ab_s, beta_s, bx_s,
          H0b, X0b, Cb, eXb, eHb, cidb, gmb,
          freqs_in, Win1, bin1, Win2, bin2, Win3, bin3,
          A1, A2, wd3, Ce0, Ce1, bea, Web, beb, WxT,
          Wna, bna, Wnb, bnb, Wh2i, bh2i,
          out):
    p = pl.program_id(0)

    iota_r = jax.lax.broadcasted_iota(jnp.int32, (NPAD, 1), 0)
    rowsel = (iota_r < p).astype(jnp.float32)
    ab1 = ab_s[p]
    ab2 = ab_s[B - 1 - p]
    abr = rowsel * ab1 + (1.0 - rowsel) * ab2
    sa = jnp.sqrt(abr)
    sb = jnp.sqrt(1.0 - abr)
    gm = gmb[0, 0, :].reshape(NPAD, 1)
    H0 = H0b[0]
    X0 = X0b[0]
    cond = Cb[0]
    eX = eXb[0]
    eH = eHb[0]

    Xn0 = gm * (sa * X0 + sb * eX) + (1.0 - gm) * X0
    Hn = gm * (sa * H0 + sb * eH) + (1.0 - gm) * H0

    freqs = freqs_in[...]
    te1 = beta_s[p] * freqs
    te2 = beta_s[B - 1 - p] * freqs
    te1 = jnp.concatenate([jnp.sin(te1), jnp.cos(te1)], axis=1)
    te2 = jnp.concatenate([jnp.sin(te2), jnp.cos(te2)], axis=1)
    te = rowsel * te1 + (1.0 - rowsel) * te2

    feat = jnp.concatenate([Hn, cond, te], axis=1)
    h = jnp.maximum(
        jnp.dot(feat, Win1[...], preferred_element_type=jnp.float32)
        + bin1[...], 0.0)
    h = jnp.maximum(
        jnp.dot(h, Win2[...], preferred_element_type=jnp.float32)
        + bin2[...], 0.0)
    h = jnp.dot(h, Win3[...], preferred_element_type=jnp.float32) + bin3[...]

    cid = cidb[0]
    CEm = (cid.reshape(NPAD, 1) == cid.reshape(1, NPAD)).astype(jnp.float32)
    jota = jax.lax.broadcasted_iota(jnp.int32, (1, NPAD), 1)
    validj = (jota < NPAD - 1).astype(jnp.float32)
    rsj = rowsel.reshape(1, NPAD)
    Mpair = (rowsel * rsj + (1.0 - rowsel) * (1.0 - rsj)) * validj
    pf = p.astype(jnp.float32)
    invr = rowsel / (pf + 1.0) + (1.0 - rowsel) / (64.0 - pf)

    Xn = Xn0
    for l in range(NLAYERS):
        P = (jnp.dot(h, A1[l], preferred_element_type=jnp.float32)
             + bea[l])
        Q = jnp.dot(h, A2[l], preferred_element_type=jnp.float32)
        n2 = jnp.sum(Xn * Xn, axis=1, keepdims=True)
        G = jnp.dot(Xn, Xn.T, preferred_element_type=jnp.float32)
        D2 = n2 + n2.reshape(1, NPAD) - 2.0 * G
        ce0 = Ce0[l]
        ce1 = Ce1[l]
        E = (P.reshape(NPAD, 1, HIDDEN)
             + Q.reshape(1, NPAD, HIDDEN)
             + D2.reshape(NPAD, NPAD, 1) * wd3[l].reshape(1, 1, HIDDEN)
             + (ce1.reshape(1, 1, HIDDEN)
                + CEm.reshape(NPAD, NPAD, 1)
                * (ce0 - ce1).reshape(1, 1, HIDDEN)))
        M1 = jnp.maximum(E, 0.0).reshape(NPAD * NPAD, HIDDEN)
        M2 = jnp.maximum(
            jnp.dot(M1, Web[l], preferred_element_type=jnp.float32)
            + beb[l], 0.0)
        xw = jnp.tanh(
            jnp.dot(M2, WxT[l], preferred_element_type=jnp.float32)
            + bx_s[l])
        M2m = M2.reshape(NPAD, NPAD, HIDDEN) * Mpair.reshape(NPAD, NPAD, 1)
        XW = xw.reshape(NPAD, NPAD) * Mpair
        agg = jnp.sum(M2m, axis=1) * invr
        S = jnp.sum(XW, axis=1, keepdims=True)
        Xn = Xn + (Xn * S
                   - jnp.dot(XW, Xn,
                             preferred_element_type=jnp.float32)) * invr
        nh = jnp.concatenate([h, agg], axis=1)
        h = (h
             + jnp.dot(
                 jnp.maximum(
                     jnp.dot(nh, Wna[l], preferred_element_type=jnp.float32)
                     + bna[l], 0.0),
                 Wnb[l], preferred_element_type=jnp.float32)
             + bnb[l])

    nH = jnp.dot(h, Wh2i[...], preferred_element_type=jnp.float32) + bh2i[...]
    exd = (Xn - Xn0) - eX
    pX = jnp.sum(jnp.sum(exd * exd, axis=1, keepdims=True) * gm)
    ehd = (nH - Hn) - eH
    pH = jnp.sum(jnp.sum(ehd * ehd, axis=1, keepdims=True) * gm)
    pD = jnp.sum(gm)
    lane = jax.lax.broadcasted_iota(jnp.int32, (1, HIDDEN), 1)
    vec = (jnp.where(lane == 0, pX, 0.0)
           + jnp.where(lane == 1, pH, 0.0)
           + jnp.where(lane == 2, pD, 0.0))
    out[...] = vec.reshape(1, 1, HIDDEN)


def kernel(H_0, X_0, cond_embedding, chain_ids, generate_mask, lengths, t,
           Win1, bin1, Win2, bin2, Win3, bin3, Wh2i, bh2i, Eemb,
           We_a, be_a, We_b, be_b, Wx, bx, Wn_a, bn_a, Wn_b, bn_b):
    kx, kh = jax.random.split(jax.random.key(42))
    eps_X = jax.random.normal(kx, X_0.shape, dtype=jnp.float32)
    eps_H = jax.random.normal(kh, H_0.shape, dtype=jnp.float32)
    betas = jnp.concatenate([jnp.zeros(1), jnp.linspace(1e-4, 0.02, NSTEPS)])
    abars = jnp.cumprod(1.0 - betas)
    ab_vec = abars[t].astype(jnp.float32)
    beta_vec = betas[t].astype(jnp.float32)

    src = jnp.asarray(_SRC_NP).reshape(-1)
    vf = jnp.asarray(_VALID_NP.astype(np.float32))

    def padc(a, c):
        ap = a[src].reshape(NT, NPAD, a.shape[1]) * vf[:, :, None]
        if a.shape[1] < c:
            ap = jnp.pad(ap, ((0, 0), (0, 0), (0, c - a.shape[1])))
        return ap

    H0p = padc(H_0, LATENT)
    X0p = padc(X_0, XPAD)
    condp = padc(cond_embedding, HIDDEN)
    eXp = padc(eps_X, XPAD)
    eHp = padc(eps_H, LATENT)
    cidp = chain_ids.astype(jnp.int32)[src].reshape(NT, 1, NPAD)
    gmp = (generate_mask.astype(jnp.float32)[src].reshape(NT, 1, NPAD)
           * vf.reshape(NT, 1, NPAD))

    A1 = We_a[:, :HIDDEN, :]
    A2 = We_a[:, HIDDEN:2 * HIDDEN, :]
    wd3 = We_a[:, 2 * HIDDEN:2 * HIDDEN + 1, :]
    CeW = jnp.einsum('ec,lch->leh', Eemb, We_a[:, 2 * HIDDEN + 1:, :])
    Ce0 = CeW[:, 0:1, :].reshape(NLAYERS, 1, HIDDEN)
    Ce1 = CeW[:, 1:2, :].reshape(NLAYERS, 1, HIDDEN)

    blk = lambda shp: pl.BlockSpec(shp, lambda b: (b, 0, 0))
    rep = lambda arr: pl.BlockSpec(arr.shape,
                                   lambda b, _n=arr.ndim: (0,) * _n)
    smem = pl.BlockSpec(memory_space=pltpu.SMEM)

    weights = [jnp.asarray(_FREQS_NP),
               Win1, bin1.reshape(1, HIDDEN), Win2, bin2.reshape(1, HIDDEN),
               Win3, bin3.reshape(1, HIDDEN),
               A1, A2, wd3, Ce0, Ce1, be_a.reshape(NLAYERS, 1, HIDDEN),
               We_b, be_b.reshape(NLAYERS, 1, HIDDEN),
               Wx,
               Wn_a, bn_a.reshape(NLAYERS, 1, HIDDEN),
               Wn_b, bn_b.reshape(NLAYERS, 1, HIDDEN),
               Wh2i, bh2i.reshape(1, LATENT)]

    res = pl.pallas_call(
        _body,
        grid=(NT,),
        in_specs=[smem, smem, smem,
                  blk((1, NPAD, LATENT)), blk((1, NPAD, XPAD)),
                  blk((1, NPAD, HIDDEN)), blk((1, NPAD, XPAD)),
                  blk((1, NPAD, LATENT)),
                  blk((1, 1, NPAD)), blk((1, 1, NPAD))]
                 + [rep(w) for w in weights],
        out_specs=pl.BlockSpec((1, 1, HIDDEN), lambda b: (b, 0, 0)),
        out_shape=jax.ShapeDtypeStruct((NT, 1, HIDDEN), jnp.float32),
        compiler_params=pltpu.CompilerParams(
            dimension_semantics=("parallel",)),
    )(ab_vec, beta_vec, bx.reshape(-1),
      H0p, X0p, condp, eXp, eHp, cidp, gmp, *weights)

    tot = jnp.sum(res, axis=(0, 1))
    denom = tot[2] + 1e-8
    return jnp.stack([tot[0] / denom, tot[1] / denom])


# speedup vs baseline: 15.1795x; 1.0151x over previous
import numpy as np
import jax
import jax.numpy as jnp
from jax.experimental import pallas as pl
from jax.experimental.pallas import tpu as pltpu

LATENT = 16
HIDDEN = 128
NSTEPS = 100
NLAYERS = 3
B = 64
NPAD = 64
XPAD = 8

NT = B // 2
_offs = (np.arange(B) * (np.arange(B) - 1)) // 2
_r = np.arange(NPAD)[None, :]
_p = np.arange(NT)[:, None]
_in1 = _r < _p
_in2 = (_r >= _p) & (_r < NPAD - 1)
_SRC_NP = np.where(
    _in1, _offs[np.minimum(_p, B - 1)] + _r,
    np.where(_in2, _offs[B - 1 - _p] + (_r - _p), 0)).astype(np.int32)
_VALID_NP = (_in1 | _in2)
_FREQS_NP = np.exp(-np.log(10000.0) * np.arange(HIDDEN // 2)
                   / (HIDDEN // 2 - 1)).astype(np.float32).reshape(1, -1)


def _body(ab_s, beta_s, bx_s,
          H0b, X0b, Cb, eXb, eHb, cidb, gmb,
          freqs_in, Win1, bin1, Win2, bin2, Win3, bin3,
          A1, A2, wd3, Dce, bea, Web, beb, WxT,
          Wna, bna, Wnb, bnb, Wh2i, bh2i,
          out):
    p = pl.program_id(0)

    iota_r = jax.lax.broadcasted_iota(jnp.int32, (NPAD, 1), 0)
    rowsel = (iota_r < p).astype(jnp.float32)
    ab1 = ab_s[p]
    ab2 = ab_s[B - 1 - p]
    abr = rowsel * ab1 + (1.0 - rowsel) * ab2
    sa = jnp.sqrt(abr)
    sb = jnp.sqrt(1.0 - abr)
    gm = gmb[0, 0, :].reshape(NPAD, 1)
    H0 = H0b[0]
    X0 = X0b[0]
    cond = Cb[0]
    eX = eXb[0]
    eH = eHb[0]

    Xn0 = gm * (sa * X0 + sb * eX) + (1.0 - gm) * X0
    Hn = gm * (sa * H0 + sb * eH) + (1.0 - gm) * H0

    freqs = freqs_in[...]
    te1 = beta_s[p] * freqs
    te2 = beta_s[B - 1 - p] * freqs
    te1 = jnp.concatenate([jnp.sin(te1), jnp.cos(te1)], axis=1)
    te2 = jnp.concatenate([jnp.sin(te2), jnp.cos(te2)], axis=1)
    te = rowsel * te1 + (1.0 - rowsel) * te2

    feat = jnp.concatenate([Hn, cond, te], axis=1)
    h = jnp.maximum(
        jnp.dot(feat, Win1[...], preferred_element_type=jnp.float32)
        + bin1[...], 0.0)
    h = jnp.maximum(
        jnp.dot(h, Win2[...], preferred_element_type=jnp.float32)
        + bin2[...], 0.0)
    h = jnp.dot(h, Win3[...], preferred_element_type=jnp.float32) + bin3[...]

    cid = cidb[0]
    CEm = (cid.reshape(NPAD, 1) == cid.reshape(1, NPAD)).astype(jnp.float32)
    jota = jax.lax.broadcasted_iota(jnp.int32, (1, NPAD), 1)
    validj = (jota < NPAD - 1).astype(jnp.float32)
    rsj = rowsel.reshape(1, NPAD)
    Mpair = (rowsel * rsj + (1.0 - rowsel) * (1.0 - rsj)) * validj
    pf = p.astype(jnp.float32)
    invr = rowsel / (pf + 1.0) + (1.0 - rowsel) / (64.0 - pf)

    Xn = Xn0
    for l in range(NLAYERS):
        P = (jnp.dot(h, A1[l], preferred_element_type=jnp.float32)
             + bea[l])
        Q = jnp.dot(h, A2[l], preferred_element_type=jnp.float32)
        n2 = jnp.sum(Xn * Xn, axis=1, keepdims=True)
        G = jnp.dot(Xn, Xn.T, preferred_element_type=jnp.float32)
        D2 = n2 + n2.reshape(1, NPAD) - 2.0 * G
        E = (P.reshape(NPAD, 1, HIDDEN)
             + Q.reshape(1, NPAD, HIDDEN)
             + D2.reshape(NPAD, NPAD, 1) * wd3[l].reshape(1, 1, HIDDEN)
             + CEm.reshape(NPAD, NPAD, 1) * Dce[l].reshape(1, 1, HIDDEN))
        M1 = jnp.maximum(E, 0.0).reshape(NPAD * NPAD, HIDDEN)
        M2 = jnp.maximum(
            jnp.dot(M1, Web[l], preferred_element_type=jnp.float32)
            + beb[l], 0.0)
        vcol = jnp.dot(M2, WxT[l],
                       preferred_element_type=jnp.float32)
        M2m = M2.reshape(NPAD, NPAD, HIDDEN) * Mpair.reshape(NPAD, NPAD, 1)
        XW = jnp.tanh(vcol.reshape(NPAD, NPAD) + bx_s[l]) * Mpair
        agg = jnp.sum(M2m, axis=1) * invr
        S = jnp.sum(XW, axis=1, keepdims=True)
        Xn = Xn + (Xn * S
                   - jnp.dot(XW, Xn,
                             preferred_element_type=jnp.float32)) * invr
        nh = jnp.concatenate([h, agg], axis=1)
        h = (h
             + jnp.dot(
                 jnp.maximum(
                     jnp.dot(nh, Wna[l], preferred_element_type=jnp.float32)
                     + bna[l], 0.0),
                 Wnb[l], preferred_element_type=jnp.float32)
             + bnb[l])

    nH = jnp.dot(h, Wh2i[...], preferred_element_type=jnp.float32) + bh2i[...]
    exd = (Xn - Xn0) - eX
    pX = jnp.sum(jnp.sum(exd * exd, axis=1, keepdims=True) * gm)
    ehd = (nH - Hn) - eH
    pH = jnp.sum(jnp.sum(ehd * ehd, axis=1, keepdims=True) * gm)
    pD = jnp.sum(gm)
    lane = jax.lax.broadcasted_iota(jnp.int32, (1, HIDDEN), 1)
    vec = (jnp.where(lane == 0, pX, 0.0)
           + jnp.where(lane == 1, pH, 0.0)
           + jnp.where(lane == 2, pD, 0.0))
    out[...] = vec.reshape(1, 1, HIDDEN)


def kernel(H_0, X_0, cond_embedding, chain_ids, generate_mask, lengths, t,
           Win1, bin1, Win2, bin2, Win3, bin3, Wh2i, bh2i, Eemb,
           We_a, be_a, We_b, be_b, Wx, bx, Wn_a, bn_a, Wn_b, bn_b):
    kx, kh = jax.random.split(jax.random.key(42))
    eps_X = jax.random.normal(kx, X_0.shape, dtype=jnp.float32)
    eps_H = jax.random.normal(kh, H_0.shape, dtype=jnp.float32)
    betas = jnp.concatenate([jnp.zeros(1), jnp.linspace(1e-4, 0.02, NSTEPS)])
    abars = jnp.cumprod(1.0 - betas)
    ab_vec = abars[t].astype(jnp.float32)
    beta_vec = betas[t].astype(jnp.float32)

    src = jnp.asarray(_SRC_NP).reshape(-1)
    vf = jnp.asarray(_VALID_NP.astype(np.float32))

    def padc(a, c):
        ap = a[src].reshape(NT, NPAD, a.shape[1]) * vf[:, :, None]
        if a.shape[1] < c:
            ap = jnp.pad(ap, ((0, 0), (0, 0), (0, c - a.shape[1])))
        return ap

    H0p = padc(H_0, LATENT)
    X0p = padc(X_0, XPAD)
    condp = padc(cond_embedding, HIDDEN)
    eXp = padc(eps_X, XPAD)
    eHp = padc(eps_H, LATENT)
    cidp = chain_ids.astype(jnp.int32)[src].reshape(NT, 1, NPAD)
    gmp = (generate_mask.astype(jnp.float32)[src].reshape(NT, 1, NPAD)
           * vf.reshape(NT, 1, NPAD))

    A1 = We_a[:, :HIDDEN, :]
    A2 = We_a[:, HIDDEN:2 * HIDDEN, :]
    wd3 = We_a[:, 2 * HIDDEN:2 * HIDDEN + 1, :]
    CeW = jnp.einsum('ec,lch->leh', Eemb, We_a[:, 2 * HIDDEN + 1:, :])
    Ce0 = CeW[:, 0:1, :].reshape(NLAYERS, 1, HIDDEN)
    Ce1 = CeW[:, 1:2, :].reshape(NLAYERS, 1, HIDDEN)
    Dce = Ce0 - Ce1
    bea2 = be_a.reshape(NLAYERS, 1, HIDDEN) + Ce1

    blk = lambda shp: pl.BlockSpec(shp, lambda b: (b, 0, 0))
    rep = lambda arr: pl.BlockSpec(arr.shape,
                                   lambda b, _n=arr.ndim: (0,) * _n)
    smem = pl.BlockSpec(memory_space=pltpu.SMEM)

    weights = [jnp.asarray(_FREQS_NP),
               Win1, bin1.reshape(1, HIDDEN), Win2, bin2.reshape(1, HIDDEN),
               Win3, bin3.reshape(1, HIDDEN),
               A1, A2, wd3, Dce, bea2,
               We_b, be_b.reshape(NLAYERS, 1, HIDDEN),
               Wx,
               Wn_a, bn_a.reshape(NLAYERS, 1, HIDDEN),
               Wn_b, bn_b.reshape(NLAYERS, 1, HIDDEN),
               Wh2i, bh2i.reshape(1, LATENT)]

    res = pl.pallas_call(
        _body,
        grid=(NT,),
        in_specs=[smem, smem, smem,
                  blk((1, NPAD, LATENT)), blk((1, NPAD, XPAD)),
                  blk((1, NPAD, HIDDEN)), blk((1, NPAD, XPAD)),
                  blk((1, NPAD, LATENT)),
                  blk((1, 1, NPAD)), blk((1, 1, NPAD))]
                 + [rep(w) for w in weights],
        out_specs=pl.BlockSpec((1, 1, HIDDEN), lambda b: (b, 0, 0)),
        out_shape=jax.ShapeDtypeStruct((NT, 1, HIDDEN), jnp.float32),
        compiler_params=pltpu.CompilerParams(
            dimension_semantics=("parallel",)),
    )(ab_vec, beta_vec, bx.reshape(-1),
      H0p, X0p, condp, eXp, eHp, cidp, gmp, *weights)

    tot = jnp.sum(res, axis=(0, 1))
    denom = tot[2] + 1e-8
    return jnp.stack([tot[0] / denom, tot[1] / denom])


# speedup vs baseline: 15.7841x; 1.0398x over previous
import numpy as np
import jax
import jax.numpy as jnp
from jax.experimental import pallas as pl
from jax.experimental.pallas import tpu as pltpu

LATENT = 16
HIDDEN = 128
NSTEPS = 100
NLAYERS = 3
B = 64
NPAD = 64
XPAD = 8

NT = B // 2
_offs = (np.arange(B) * (np.arange(B) - 1)) // 2
_r = np.arange(NPAD)[None, :]
_p = np.arange(NT)[:, None]
_in1 = _r < _p
_in2 = (_r >= _p) & (_r < NPAD - 1)
_SRC_NP = np.where(
    _in1, _offs[np.minimum(_p, B - 1)] + _r,
    np.where(_in2, _offs[B - 1 - _p] + (_r - _p), 0)).astype(np.int32)
_VALID_NP = (_in1 | _in2)
_FREQS_NP = np.exp(-np.log(10000.0) * np.arange(HIDDEN // 2)
                   / (HIDDEN // 2 - 1)).astype(np.float32).reshape(1, -1)


def _body(ab_s, beta_s, bx_s,
          H0b, X0b, Cb, eXb, eHb, cidb, gmb,
          freqs_in, Win1, bin1, Win2, bin2, Win3, bin3,
          A1, A2, wd3, Dce, bea, Web, beb, WxT,
          Wna, bna, Wnb, bnb, Wh2i, bh2i,
          out):
    p = pl.program_id(0)

    iota_r = jax.lax.broadcasted_iota(jnp.int32, (NPAD, 1), 0)
    rb = iota_r < p
    rowsel = rb.astype(jnp.float32)
    ab1 = ab_s[p]
    ab2 = ab_s[B - 1 - p]
    abr = rowsel * ab1 + (1.0 - rowsel) * ab2
    sa = jnp.sqrt(abr)
    sb = jnp.sqrt(1.0 - abr)
    gm = gmb[0, 0, :].reshape(NPAD, 1)

    o1 = (p * (p - 1)) // 2
    q = B - 1 - p
    o2 = (q * (q - 1)) // 2 + q - NPAD

    def win(ref):
        w1 = ref[pl.ds(o1, NPAD), :]
        w2 = ref[pl.ds(o2, NPAD), :]
        w2s = jnp.concatenate([w2[1:NPAD], w2[0:1]], axis=0)
        return jnp.where(rb, w1, w2s)

    H0 = win(H0b)
    X0 = win(X0b)
    cond = win(Cb)
    eX = win(eXb)
    eH = win(eHb)

    Xn0 = gm * (sa * X0 + sb * eX) + (1.0 - gm) * X0
    Hn = gm * (sa * H0 + sb * eH) + (1.0 - gm) * H0

    freqs = freqs_in[...]
    te1 = beta_s[p] * freqs
    te2 = beta_s[B - 1 - p] * freqs
    te1 = jnp.concatenate([jnp.sin(te1), jnp.cos(te1)], axis=1)
    te2 = jnp.concatenate([jnp.sin(te2), jnp.cos(te2)], axis=1)
    te = rowsel * te1 + (1.0 - rowsel) * te2

    feat = jnp.concatenate([Hn, cond, te], axis=1)
    h = jnp.maximum(
        jnp.dot(feat, Win1[...], preferred_element_type=jnp.float32)
        + bin1[...], 0.0)
    h = jnp.maximum(
        jnp.dot(h, Win2[...], preferred_element_type=jnp.float32)
        + bin2[...], 0.0)
    h = jnp.dot(h, Win3[...], preferred_element_type=jnp.float32) + bin3[...]

    cid = cidb[0]
    CEm = (cid.reshape(NPAD, 1) == cid.reshape(1, NPAD)).astype(jnp.float32)
    jota = jax.lax.broadcasted_iota(jnp.int32, (1, NPAD), 1)
    validj = (jota < NPAD - 1).astype(jnp.float32)
    rsj = rowsel.reshape(1, NPAD)
    Mpair = (rowsel * rsj + (1.0 - rowsel) * (1.0 - rsj)) * validj
    pf = p.astype(jnp.float32)
    invr = rowsel / (pf + 1.0) + (1.0 - rowsel) / (64.0 - pf)

    Xn = Xn0
    for l in range(NLAYERS):
        P = (jnp.dot(h, A1[l], preferred_element_type=jnp.float32)
             + bea[l])
        Q = jnp.dot(h, A2[l], preferred_element_type=jnp.float32)
        n2 = jnp.sum(Xn * Xn, axis=1, keepdims=True)
        G = jnp.dot(Xn, Xn.T, preferred_element_type=jnp.float32)
        D2 = n2 + n2.reshape(1, NPAD) - 2.0 * G
        E = (P.reshape(NPAD, 1, HIDDEN)
             + Q.reshape(1, NPAD, HIDDEN)
             + D2.reshape(NPAD, NPAD, 1) * wd3[l].reshape(1, 1, HIDDEN)
             + CEm.reshape(NPAD, NPAD, 1) * Dce[l].reshape(1, 1, HIDDEN))
        M1 = jnp.maximum(E, 0.0).reshape(NPAD * NPAD, HIDDEN)
        M2 = jnp.maximum(
            jnp.dot(M1, Web[l], preferred_element_type=jnp.float32)
            + beb[l], 0.0)
        vcol = jnp.dot(M2, WxT[l],
                       preferred_element_type=jnp.float32)
        M2m = M2.reshape(NPAD, NPAD, HIDDEN) * Mpair.reshape(NPAD, NPAD, 1)
        XW = jnp.tanh(vcol.reshape(NPAD, NPAD) + bx_s[l]) * Mpair
        agg = jnp.sum(M2m, axis=1) * invr
        S = jnp.sum(XW, axis=1, keepdims=True)
        Xn = Xn + (Xn * S
                   - jnp.dot(XW, Xn,
                             preferred_element_type=jnp.float32)) * invr
        nh = jnp.concatenate([h, agg], axis=1)
        h = (h
             + jnp.dot(
                 jnp.maximum(
                     jnp.dot(nh, Wna[l], preferred_element_type=jnp.float32)
                     + bna[l], 0.0),
                 Wnb[l], preferred_element_type=jnp.float32)
             + bnb[l])

    nH = jnp.dot(h, Wh2i[...], preferred_element_type=jnp.float32) + bh2i[...]
    exd = (Xn - Xn0) - eX
    pX = jnp.sum(jnp.sum(exd * exd, axis=1, keepdims=True) * gm)
    ehd = (nH - Hn) - eH
    pH = jnp.sum(jnp.sum(ehd * ehd, axis=1, keepdims=True) * gm)
    pD = jnp.sum(gm)
    lane = jax.lax.broadcasted_iota(jnp.int32, (1, HIDDEN), 1)
    vec = (jnp.where(lane == 0, pX, 0.0)
           + jnp.where(lane == 1, pH, 0.0)
           + jnp.where(lane == 2, pD, 0.0))
    out[...] = vec.reshape(1, 1, HIDDEN)


def kernel(H_0, X_0, cond_embedding, chain_ids, generate_mask, lengths, t,
           Win1, bin1, Win2, bin2, Win3, bin3, Wh2i, bh2i, Eemb,
           We_a, be_a, We_b, be_b, Wx, bx, Wn_a, bn_a, Wn_b, bn_b):
    kx, kh = jax.random.split(jax.random.key(42))
    eps_X = jax.random.normal(kx, X_0.shape, dtype=jnp.float32)
    eps_H = jax.random.normal(kh, H_0.shape, dtype=jnp.float32)
    betas = jnp.concatenate([jnp.zeros(1), jnp.linspace(1e-4, 0.02, NSTEPS)])
    abars = jnp.cumprod(1.0 - betas)
    ab_vec = abars[t].astype(jnp.float32)
    beta_vec = betas[t].astype(jnp.float32)

    src = jnp.asarray(_SRC_NP).reshape(-1)
    vf = jnp.asarray(_VALID_NP.astype(np.float32))

    H0p = H_0
    X0p = jnp.pad(X_0, ((0, 0), (0, XPAD - 3)))
    condp = cond_embedding
    eXp = jnp.pad(eps_X, ((0, 0), (0, XPAD - 3)))
    eHp = eps_H
    cidp = chain_ids.astype(jnp.int32)[src].reshape(NT, 1, NPAD)
    gmp = (generate_mask.astype(jnp.float32)[src].reshape(NT, 1, NPAD)
           * vf.reshape(NT, 1, NPAD))

    A1 = We_a[:, :HIDDEN, :]
    A2 = We_a[:, HIDDEN:2 * HIDDEN, :]
    wd3 = We_a[:, 2 * HIDDEN:2 * HIDDEN + 1, :]
    CeW = jnp.einsum('ec,lch->leh', Eemb, We_a[:, 2 * HIDDEN + 1:, :])
    Ce0 = CeW[:, 0:1, :].reshape(NLAYERS, 1, HIDDEN)
    Ce1 = CeW[:, 1:2, :].reshape(NLAYERS, 1, HIDDEN)
    Dce = Ce0 - Ce1
    bea2 = be_a.reshape(NLAYERS, 1, HIDDEN) + Ce1

    blk = lambda shp: pl.BlockSpec(shp, lambda b: (b, 0, 0))
    rep = lambda arr: pl.BlockSpec(arr.shape,
                                   lambda b, _n=arr.ndim: (0,) * _n)
    smem = pl.BlockSpec(memory_space=pltpu.SMEM)

    weights = [jnp.asarray(_FREQS_NP),
               Win1, bin1.reshape(1, HIDDEN), Win2, bin2.reshape(1, HIDDEN),
               Win3, bin3.reshape(1, HIDDEN),
               A1, A2, wd3, Dce, bea2,
               We_b, be_b.reshape(NLAYERS, 1, HIDDEN),
               Wx,
               Wn_a, bn_a.reshape(NLAYERS, 1, HIDDEN),
               Wn_b, bn_b.reshape(NLAYERS, 1, HIDDEN),
               Wh2i, bh2i.reshape(1, LATENT)]

    res = pl.pallas_call(
        _body,
        grid=(NT,),
        in_specs=[smem, smem, smem,
                  rep(H0p), rep(X0p), rep(condp), rep(eXp), rep(eHp),
                  blk((1, 1, NPAD)), blk((1, 1, NPAD))]
                 + [rep(w) for w in weights],
        out_specs=pl.BlockSpec((1, 1, HIDDEN), lambda b: (b, 0, 0)),
        out_shape=jax.ShapeDtypeStruct((NT, 1, HIDDEN), jnp.float32),
        compiler_params=pltpu.CompilerParams(
            dimension_semantics=("parallel",)),
    )(ab_vec, beta_vec, bx.reshape(-1),
      H0p, X0p, condp, eXp, eHp, cidp, gmp, *weights)

    tot = jnp.sum(res, axis=(0, 1))
    denom = tot[2] + 1e-8
    return jnp.stack([tot[0] / denom, tot[1] / denom])


# speedup vs baseline: 18.2389x; 1.1555x over previous
import numpy as np
import jax
import jax.numpy as jnp
from jax.experimental import pallas as pl
from jax.experimental.pallas import tpu as pltpu

LATENT = 16
HIDDEN = 128
NSTEPS = 100
NLAYERS = 3
B = 64
NPAD = 64
XPAD = 8

NT = B // 2
_offs = (np.arange(B) * (np.arange(B) - 1)) // 2
_r = np.arange(NPAD)[None, :]
_p = np.arange(NT)[:, None]
_in1 = _r < _p
_in2 = (_r >= _p) & (_r < NPAD - 1)
_SRC_NP = np.where(
    _in1, _offs[np.minimum(_p, B - 1)] + _r,
    np.where(_in2, _offs[B - 1 - _p] + (_r - _p), 0)).astype(np.int32)
_VALID_NP = (_in1 | _in2)
_FREQS_NP = np.exp(-np.log(10000.0) * np.arange(HIDDEN // 2)
                   / (HIDDEN // 2 - 1)).astype(np.float32).reshape(1, -1)


def _body(ab_s, beta_s, bx_s,
          H0b, X0b, Cb, eXb, eHb, cidb, gmb,
          freqs_in, Win1, bin1, Win2, bin2, Win3, bin3,
          A1, A2, wd3, Dce, bea, Web, beb, WxT,
          Wna, bna, Wnb, bnb, Wh2i, bh2i,
          out):
    p = pl.program_id(0)

    iota_r = jax.lax.broadcasted_iota(jnp.int32, (NPAD, 1), 0)
    rb = iota_r < p
    rowsel = rb.astype(jnp.float32)
    ab1 = ab_s[p]
    ab2 = ab_s[B - 1 - p]
    abr = rowsel * ab1 + (1.0 - rowsel) * ab2
    sa = jnp.sqrt(abr)
    sb = jnp.sqrt(1.0 - abr)
    gm = gmb[0, 0, :].reshape(NPAD, 1)

    o1 = (p * (p - 1)) // 2
    q = B - 1 - p
    o2 = (q * (q - 1)) // 2 + q - NPAD

    def win(ref):
        w1 = ref[pl.ds(o1, NPAD), :]
        w2 = ref[pl.ds(o2, NPAD), :]
        w2s = jnp.concatenate([w2[1:NPAD], w2[0:1]], axis=0)
        return jnp.where(rb, w1, w2s)

    H0 = win(H0b)
    X0 = win(X0b)
    cond = win(Cb)
    eX = win(eXb)
    eH = win(eHb)

    Xn0 = gm * (sa * X0 + sb * eX) + (1.0 - gm) * X0
    Hn = gm * (sa * H0 + sb * eH) + (1.0 - gm) * H0

    freqs = freqs_in[...]
    te1 = beta_s[p] * freqs
    te2 = beta_s[B - 1 - p] * freqs
    te1 = jnp.concatenate([jnp.sin(te1), jnp.cos(te1)], axis=1)
    te2 = jnp.concatenate([jnp.sin(te2), jnp.cos(te2)], axis=1)
    te = rowsel * te1 + (1.0 - rowsel) * te2

    feat = jnp.concatenate([Hn, cond, te], axis=1)
    h = jnp.maximum(
        jnp.dot(feat, Win1[...], preferred_element_type=jnp.float32)
        + bin1[...], 0.0)
    h = jnp.maximum(
        jnp.dot(h, Win2[...], preferred_element_type=jnp.float32)
        + bin2[...], 0.0)
    h = jnp.dot(h, Win3[...], preferred_element_type=jnp.float32) + bin3[...]

    cid = cidb[0]
    CEm = (cid.reshape(NPAD, 1) == cid.reshape(1, NPAD)).astype(jnp.float32)
    jota = jax.lax.broadcasted_iota(jnp.int32, (1, NPAD), 1)
    validj = (jota < NPAD - 1).astype(jnp.float32)
    rsj = rowsel.reshape(1, NPAD)
    Mpair = (rowsel * rsj + (1.0 - rowsel) * (1.0 - rsj)) * validj
    pf = p.astype(jnp.float32)
    invr = rowsel / (pf + 1.0) + (1.0 - rowsel) / (64.0 - pf)

    Xn = Xn0
    for l in range(NLAYERS):
        P = (jnp.dot(h, A1[l], preferred_element_type=jnp.float32)
             + bea[l])
        Q = jnp.dot(h, A2[l], preferred_element_type=jnp.float32)
        n2 = jnp.sum(Xn * Xn, axis=1, keepdims=True)
        G = jnp.dot(Xn, Xn.T, preferred_element_type=jnp.float32)
        D2 = n2 + n2.reshape(1, NPAD) - 2.0 * G
        Pb = P.astype(jnp.bfloat16)
        Qb = Q.astype(jnp.bfloat16)
        D2b = D2.astype(jnp.bfloat16)
        CEb = CEm.astype(jnp.bfloat16)
        E = (Pb.reshape(NPAD, 1, HIDDEN)
             + Qb.reshape(1, NPAD, HIDDEN)
             + D2b.reshape(NPAD, NPAD, 1) * wd3[l].reshape(1, 1, HIDDEN)
             + CEb.reshape(NPAD, NPAD, 1) * Dce[l].reshape(1, 1, HIDDEN))
        M1 = jnp.maximum(E, jnp.bfloat16(0.0)).reshape(NPAD * NPAD, HIDDEN)
        M2 = jnp.maximum(
            jnp.dot(M1, Web[l], preferred_element_type=jnp.float32)
            + beb[l], 0.0)
        vcol = jnp.dot(M2, WxT[l],
                       preferred_element_type=jnp.float32)
        M2m = M2.reshape(NPAD, NPAD, HIDDEN) * Mpair.reshape(NPAD, NPAD, 1)
        XW = jnp.tanh(vcol.reshape(NPAD, NPAD) + bx_s[l]) * Mpair
        agg = jnp.sum(M2m, axis=1) * invr
        S = jnp.sum(XW, axis=1, keepdims=True)
        Xn = Xn + (Xn * S
                   - jnp.dot(XW, Xn,
                             preferred_element_type=jnp.float32)) * invr
        nh = jnp.concatenate([h, agg], axis=1)
        h = (h
             + jnp.dot(
                 jnp.maximum(
                     jnp.dot(nh, Wna[l], preferred_element_type=jnp.float32)
                     + bna[l], 0.0),
                 Wnb[l], preferred_element_type=jnp.float32)
             + bnb[l])

    nH = jnp.dot(h, Wh2i[...], preferred_element_type=jnp.float32) + bh2i[...]
    exd = (Xn - Xn0) - eX
    pX = jnp.sum(jnp.sum(exd * exd, axis=1, keepdims=True) * gm)
    ehd = (nH - Hn) - eH
    pH = jnp.sum(jnp.sum(ehd * ehd, axis=1, keepdims=True) * gm)
    pD = jnp.sum(gm)
    lane = jax.lax.broadcasted_iota(jnp.int32, (1, HIDDEN), 1)
    vec = (jnp.where(lane == 0, pX, 0.0)
           + jnp.where(lane == 1, pH, 0.0)
           + jnp.where(lane == 2, pD, 0.0))
    out[...] = vec.reshape(1, 1, HIDDEN)


def kernel(H_0, X_0, cond_embedding, chain_ids, generate_mask, lengths, t,
           Win1, bin1, Win2, bin2, Win3, bin3, Wh2i, bh2i, Eemb,
           We_a, be_a, We_b, be_b, Wx, bx, Wn_a, bn_a, Wn_b, bn_b):
    kx, kh = jax.random.split(jax.random.key(42))
    eps_X = jax.random.normal(kx, X_0.shape, dtype=jnp.float32)
    eps_H = jax.random.normal(kh, H_0.shape, dtype=jnp.float32)
    betas = jnp.concatenate([jnp.zeros(1), jnp.linspace(1e-4, 0.02, NSTEPS)])
    abars = jnp.cumprod(1.0 - betas)
    ab_vec = abars[t].astype(jnp.float32)
    beta_vec = betas[t].astype(jnp.float32)

    src = jnp.asarray(_SRC_NP).reshape(-1)
    vf = jnp.asarray(_VALID_NP.astype(np.float32))

    H0p = H_0
    X0p = jnp.pad(X_0, ((0, 0), (0, XPAD - 3)))
    condp = cond_embedding
    eXp = jnp.pad(eps_X, ((0, 0), (0, XPAD - 3)))
    eHp = eps_H
    cidp = chain_ids.astype(jnp.int32)[src].reshape(NT, 1, NPAD)
    gmp = (generate_mask.astype(jnp.float32)[src].reshape(NT, 1, NPAD)
           * vf.reshape(NT, 1, NPAD))

    A1 = We_a[:, :HIDDEN, :]
    A2 = We_a[:, HIDDEN:2 * HIDDEN, :]
    wd3 = We_a[:, 2 * HIDDEN:2 * HIDDEN + 1, :]
    CeW = jnp.einsum('ec,lch->leh', Eemb, We_a[:, 2 * HIDDEN + 1:, :])
    Ce0 = CeW[:, 0:1, :].reshape(NLAYERS, 1, HIDDEN)
    Ce1 = CeW[:, 1:2, :].reshape(NLAYERS, 1, HIDDEN)
    Dce = Ce0 - Ce1
    bea2 = be_a.reshape(NLAYERS, 1, HIDDEN) + Ce1

    blk = lambda shp: pl.BlockSpec(shp, lambda b: (b, 0, 0))
    rep = lambda arr: pl.BlockSpec(arr.shape,
                                   lambda b, _n=arr.ndim: (0,) * _n)
    smem = pl.BlockSpec(memory_space=pltpu.SMEM)

    weights = [jnp.asarray(_FREQS_NP),
               Win1, bin1.reshape(1, HIDDEN), Win2, bin2.reshape(1, HIDDEN),
               Win3, bin3.reshape(1, HIDDEN),
               A1, A2, wd3.astype(jnp.bfloat16), Dce.astype(jnp.bfloat16),
               bea2,
               We_b.astype(jnp.bfloat16), be_b.reshape(NLAYERS, 1, HIDDEN),
               Wx,
               Wn_a, bn_a.reshape(NLAYERS, 1, HIDDEN),
               Wn_b, bn_b.reshape(NLAYERS, 1, HIDDEN),
               Wh2i, bh2i.reshape(1, LATENT)]

    res = pl.pallas_call(
        _body,
        grid=(NT,),
        in_specs=[smem, smem, smem,
                  rep(H0p), rep(X0p), rep(condp), rep(eXp), rep(eHp),
                  blk((1, 1, NPAD)), blk((1, 1, NPAD))]
                 + [rep(w) for w in weights],
        out_specs=pl.BlockSpec((1, 1, HIDDEN), lambda b: (b, 0, 0)),
        out_shape=jax.ShapeDtypeStruct((NT, 1, HIDDEN), jnp.float32),
        compiler_params=pltpu.CompilerParams(
            dimension_semantics=("parallel",)),
    )(ab_vec, beta_vec, bx.reshape(-1),
      H0p, X0p, condp, eXp, eHp, cidp, gmp, *weights)

    tot = jnp.sum(res, axis=(0, 1))
    denom = tot[2] + 1e-8
    return jnp.stack([tot[0] / denom, tot[1] / denom])


# speedup vs baseline: 20.2028x; 1.1077x over previous
import numpy as np
import jax
import jax.numpy as jnp
from jax.experimental import pallas as pl
from jax.experimental.pallas import tpu as pltpu

LATENT = 16
HIDDEN = 128
NSTEPS = 100
NLAYERS = 3
B = 64
NPAD = 64
XPAD = 8

NT = B // 2
_offs = (np.arange(B) * (np.arange(B) - 1)) // 2
_r = np.arange(NPAD)[None, :]
_p = np.arange(NT)[:, None]
_in1 = _r < _p
_in2 = (_r >= _p) & (_r < NPAD - 1)
_SRC_NP = np.where(
    _in1, _offs[np.minimum(_p, B - 1)] + _r,
    np.where(_in2, _offs[B - 1 - _p] + (_r - _p), 0)).astype(np.int32)
_VALID_NP = (_in1 | _in2)
_FREQS_NP = np.exp(-np.log(10000.0) * np.arange(HIDDEN // 2)
                   / (HIDDEN // 2 - 1)).astype(np.float32).reshape(1, -1)


def _body(ab_s, beta_s, bx_s,
          H0b, X0b, Cb, eXb, eHb, cidb, gmb,
          freqs_in, Win1, bin1, Win2, bin2, Win3, bin3,
          A1, A2, wd3, Dce, bea, Web, beb, WxT,
          Wna, bna, Wnb, bnb, Wh2i, bh2i,
          out, xw_scr):
    p = pl.program_id(0)

    iota_r = jax.lax.broadcasted_iota(jnp.int32, (NPAD, 1), 0)
    rb = iota_r < p
    rowsel = rb.astype(jnp.float32)
    ab1 = ab_s[p]
    ab2 = ab_s[B - 1 - p]
    abr = rowsel * ab1 + (1.0 - rowsel) * ab2
    sa = jnp.sqrt(abr)
    sb = jnp.sqrt(1.0 - abr)
    gm = gmb[0, 0, :].reshape(NPAD, 1)

    o1 = (p * (p - 1)) // 2
    q = B - 1 - p
    o2 = (q * (q - 1)) // 2 + q - NPAD

    def win(ref):
        w1 = ref[pl.ds(o1, NPAD), :]
        w2 = ref[pl.ds(o2, NPAD), :]
        w2s = jnp.concatenate([w2[1:NPAD], w2[0:1]], axis=0)
        return jnp.where(rb, w1, w2s)

    H0 = win(H0b)
    X0 = win(X0b)
    cond = win(Cb)
    eX = win(eXb)
    eH = win(eHb)

    Xn0 = gm * (sa * X0 + sb * eX) + (1.0 - gm) * X0
    Hn = gm * (sa * H0 + sb * eH) + (1.0 - gm) * H0

    freqs = freqs_in[...]
    te1 = beta_s[p] * freqs
    te2 = beta_s[B - 1 - p] * freqs
    te1 = jnp.concatenate([jnp.sin(te1), jnp.cos(te1)], axis=1)
    te2 = jnp.concatenate([jnp.sin(te2), jnp.cos(te2)], axis=1)
    te = rowsel * te1 + (1.0 - rowsel) * te2

    feat = jnp.concatenate([Hn, cond, te], axis=1)
    h = jnp.maximum(
        jnp.dot(feat, Win1[...], preferred_element_type=jnp.float32)
        + bin1[...], 0.0)
    h = jnp.maximum(
        jnp.dot(h, Win2[...], preferred_element_type=jnp.float32)
        + bin2[...], 0.0)
    h = jnp.dot(h, Win3[...], preferred_element_type=jnp.float32) + bin3[...]

    cid = cidb[0]
    CEm = (cid.reshape(NPAD, 1) == cid.reshape(1, NPAD)).astype(jnp.float32)
    jota = jax.lax.broadcasted_iota(jnp.int32, (1, NPAD), 1)
    validj = (jota < NPAD - 1).astype(jnp.float32)
    rsj = rowsel.reshape(1, NPAD)
    Mpair = (rowsel * rsj + (1.0 - rowsel) * (1.0 - rsj)) * validj
    pf = p.astype(jnp.float32)
    invr = rowsel / (pf + 1.0) + (1.0 - rowsel) / (64.0 - pf)

    Xn = Xn0
    for l in range(NLAYERS):
        P = (jnp.dot(h, A1[l], preferred_element_type=jnp.float32)
             + bea[l])
        Q = jnp.dot(h, A2[l], preferred_element_type=jnp.float32)
        n2 = jnp.sum(Xn * Xn, axis=1, keepdims=True)
        G = jnp.dot(Xn, Xn.T, preferred_element_type=jnp.float32)
        D2 = n2 + n2.reshape(1, NPAD) - 2.0 * G
        Pb = P.astype(jnp.bfloat16)
        Qb = Q.astype(jnp.bfloat16)
        D2b = D2.astype(jnp.bfloat16)
        CEb = CEm.astype(jnp.bfloat16)
        E = (Pb.reshape(NPAD, 1, HIDDEN)
             + Qb.reshape(1, NPAD, HIDDEN)
             + D2b.reshape(NPAD, NPAD, 1) * wd3[l].reshape(1, 1, HIDDEN)
             + CEb.reshape(NPAD, NPAD, 1) * Dce[l].reshape(1, 1, HIDDEN))
        M1 = jnp.maximum(E, jnp.bfloat16(0.0)).reshape(NPAD * NPAD, HIDDEN)
        M2 = jnp.maximum(
            jnp.dot(M1, Web[l], preferred_element_type=jnp.float32)
            + beb[l], 0.0)
        vcol = jnp.dot(M2, WxT[l],
                       preferred_element_type=jnp.float32)
        M2m = M2.reshape(NPAD, NPAD, HIDDEN) * Mpair.reshape(NPAD, NPAD, 1)
        xw_scr[...] = vcol.reshape(NPAD, NPAD)
        XW = jnp.tanh(xw_scr[...] + bx_s[l]) * Mpair
        agg = jnp.sum(M2m, axis=1) * invr
        S = jnp.sum(XW, axis=1, keepdims=True)
        Xn = Xn + (Xn * S
                   - jnp.dot(XW, Xn,
                             preferred_element_type=jnp.float32)) * invr
        nh = jnp.concatenate([h, agg], axis=1)
        h = (h
             + jnp.dot(
                 jnp.maximum(
                     jnp.dot(nh, Wna[l], preferred_element_type=jnp.float32)
                     + bna[l], 0.0),
                 Wnb[l], preferred_element_type=jnp.float32)
             + bnb[l])

    nH = jnp.dot(h, Wh2i[...], preferred_element_type=jnp.float32) + bh2i[...]
    exd = (Xn - Xn0) - eX
    pX = jnp.sum(jnp.sum(exd * exd, axis=1, keepdims=True) * gm)
    ehd = (nH - Hn) - eH
    pH = jnp.sum(jnp.sum(ehd * ehd, axis=1, keepdims=True) * gm)
    pD = jnp.sum(gm)
    lane = jax.lax.broadcasted_iota(jnp.int32, (1, HIDDEN), 1)
    vec = (jnp.where(lane == 0, pX, 0.0)
           + jnp.where(lane == 1, pH, 0.0)
           + jnp.where(lane == 2, pD, 0.0))
    out[...] = vec.reshape(1, 1, HIDDEN)


def kernel(H_0, X_0, cond_embedding, chain_ids, generate_mask, lengths, t,
           Win1, bin1, Win2, bin2, Win3, bin3, Wh2i, bh2i, Eemb,
           We_a, be_a, We_b, be_b, Wx, bx, Wn_a, bn_a, Wn_b, bn_b):
    kx, kh = jax.random.split(jax.random.key(42))
    eps_X = jax.random.normal(kx, X_0.shape, dtype=jnp.float32)
    eps_H = jax.random.normal(kh, H_0.shape, dtype=jnp.float32)
    betas = jnp.concatenate([jnp.zeros(1), jnp.linspace(1e-4, 0.02, NSTEPS)])
    abars = jnp.cumprod(1.0 - betas)
    ab_vec = abars[t].astype(jnp.float32)
    beta_vec = betas[t].astype(jnp.float32)

    src = jnp.asarray(_SRC_NP).reshape(-1)
    vf = jnp.asarray(_VALID_NP.astype(np.float32))

    H0p = H_0
    X0p = jnp.pad(X_0, ((0, 0), (0, XPAD - 3)))
    condp = cond_embedding
    eXp = jnp.pad(eps_X, ((0, 0), (0, XPAD - 3)))
    eHp = eps_H
    cidp = chain_ids.astype(jnp.int32)[src].reshape(NT, 1, NPAD)
    gmp = (generate_mask.astype(jnp.float32)[src].reshape(NT, 1, NPAD)
           * vf.reshape(NT, 1, NPAD))

    A1 = We_a[:, :HIDDEN, :]
    A2 = We_a[:, HIDDEN:2 * HIDDEN, :]
    wd3 = We_a[:, 2 * HIDDEN:2 * HIDDEN + 1, :]
    CeW = jnp.einsum('ec,lch->leh', Eemb, We_a[:, 2 * HIDDEN + 1:, :])
    Ce0 = CeW[:, 0:1, :].reshape(NLAYERS, 1, HIDDEN)
    Ce1 = CeW[:, 1:2, :].reshape(NLAYERS, 1, HIDDEN)
    Dce = Ce0 - Ce1
    bea2 = be_a.reshape(NLAYERS, 1, HIDDEN) + Ce1

    blk = lambda shp: pl.BlockSpec(shp, lambda b: (b, 0, 0))
    rep = lambda arr: pl.BlockSpec(arr.shape,
                                   lambda b, _n=arr.ndim: (0,) * _n)
    smem = pl.BlockSpec(memory_space=pltpu.SMEM)

    weights = [jnp.asarray(_FREQS_NP),
               Win1, bin1.reshape(1, HIDDEN), Win2, bin2.reshape(1, HIDDEN),
               Win3, bin3.reshape(1, HIDDEN),
               A1, A2, wd3.astype(jnp.bfloat16), Dce.astype(jnp.bfloat16),
               bea2,
               We_b.astype(jnp.bfloat16), be_b.reshape(NLAYERS, 1, HIDDEN),
               Wx,
               Wn_a, bn_a.reshape(NLAYERS, 1, HIDDEN),
               Wn_b, bn_b.reshape(NLAYERS, 1, HIDDEN),
               Wh2i, bh2i.reshape(1, LATENT)]

    res = pl.pallas_call(
        _body,
        grid=(NT,),
        in_specs=[smem, smem, smem,
                  rep(H0p), rep(X0p), rep(condp), rep(eXp), rep(eHp),
                  blk((1, 1, NPAD)), blk((1, 1, NPAD))]
                 + [rep(w) for w in weights],
        out_specs=pl.BlockSpec((1, 1, HIDDEN), lambda b: (b, 0, 0)),
        out_shape=jax.ShapeDtypeStruct((NT, 1, HIDDEN), jnp.float32),
        scratch_shapes=[pltpu.VMEM((NPAD, NPAD), jnp.float32)],
        compiler_params=pltpu.CompilerParams(
            dimension_semantics=("parallel",)),
    )(ab_vec, beta_vec, bx.reshape(-1),
      H0p, X0p, condp, eXp, eHp, cidp, gmp, *weights)

    tot = jnp.sum(res, axis=(0, 1))
    denom = tot[2] + 1e-8
    return jnp.stack([tot[0] / denom, tot[1] / denom])


# speedup vs baseline: 21.0358x; 1.0412x over previous
import numpy as np
import jax
import jax.numpy as jnp
from jax.experimental import pallas as pl
from jax.experimental.pallas import tpu as pltpu

LATENT = 16
HIDDEN = 128
NSTEPS = 100
NLAYERS = 3
B = 64
NPAD = 64
XPAD = 8

NT = B // 2
_offs = (np.arange(B) * (np.arange(B) - 1)) // 2
_r = np.arange(NPAD)[None, :]
_p = np.arange(NT)[:, None]
_in1 = _r < _p
_in2 = (_r >= _p) & (_r < NPAD - 1)
_SRC_NP = np.where(
    _in1, _offs[np.minimum(_p, B - 1)] + _r,
    np.where(_in2, _offs[B - 1 - _p] + (_r - _p), 0)).astype(np.int32)
_VALID_NP = (_in1 | _in2)
_FREQS_NP = np.exp(-np.log(10000.0) * np.arange(HIDDEN // 2)
                   / (HIDDEN // 2 - 1)).astype(np.float32).reshape(1, -1)


def _body(ab_s, beta_s, bx_s,
          H0b, X0b, Cb, eXb, eHb, cidb, gmb,
          freqs_in, Win1, bin1, Win2, bin2, Win3, bin3,
          A1, A2, wd3, Dce, bea, Web, beb, WxT,
          Wna, bna, Wnb, bnb, Wh2i, bh2i,
          out, xw_scr):
    p = pl.program_id(0)

    iota_r = jax.lax.broadcasted_iota(jnp.int32, (NPAD, 1), 0)
    rb = iota_r < p
    rowsel = rb.astype(jnp.float32)
    ab1 = ab_s[p]
    ab2 = ab_s[B - 1 - p]
    abr = rowsel * ab1 + (1.0 - rowsel) * ab2
    sa = jnp.sqrt(abr)
    sb = jnp.sqrt(1.0 - abr)
    gm = gmb[0, 0, :].reshape(NPAD, 1)

    o1 = (p * (p - 1)) // 2
    q = B - 1 - p
    o2 = (q * (q - 1)) // 2 + q - NPAD

    def win(ref):
        w1 = ref[pl.ds(o1, NPAD), :]
        w2 = ref[pl.ds(o2, NPAD), :]
        w2s = jnp.concatenate([w2[1:NPAD], w2[0:1]], axis=0)
        return jnp.where(rb, w1, w2s)

    H0 = win(H0b)
    X0 = win(X0b)
    cond = win(Cb)
    eX = win(eXb)
    eH = win(eHb)

    Xn0 = gm * (sa * X0 + sb * eX) + (1.0 - gm) * X0
    Hn = gm * (sa * H0 + sb * eH) + (1.0 - gm) * H0

    freqs = freqs_in[...]
    te1 = beta_s[p] * freqs
    te2 = beta_s[B - 1 - p] * freqs
    te1 = jnp.concatenate([jnp.sin(te1), jnp.cos(te1)], axis=1)
    te2 = jnp.concatenate([jnp.sin(te2), jnp.cos(te2)], axis=1)
    te = rowsel * te1 + (1.0 - rowsel) * te2

    feat = jnp.concatenate([Hn, cond, te], axis=1)
    h = jnp.maximum(
        jnp.dot(feat, Win1[...], preferred_element_type=jnp.float32)
        + bin1[...], 0.0)
    h = jnp.maximum(
        jnp.dot(h, Win2[...], preferred_element_type=jnp.float32)
        + bin2[...], 0.0)
    h = jnp.dot(h, Win3[...], preferred_element_type=jnp.float32) + bin3[...]

    cid = cidb[0]
    CEm = (cid.reshape(NPAD, 1) == cid.reshape(1, NPAD)).astype(jnp.float32)
    jota = jax.lax.broadcasted_iota(jnp.int32, (1, NPAD), 1)
    validj = (jota < NPAD - 1).astype(jnp.float32)
    rsj = rowsel.reshape(1, NPAD)
    Mpair = (rowsel * rsj + (1.0 - rowsel) * (1.0 - rsj)) * validj
    pf = p.astype(jnp.float32)
    invr = rowsel / (pf + 1.0) + (1.0 - rowsel) / (64.0 - pf)

    Mpair_bf = Mpair.astype(jnp.bfloat16)
    Xn = Xn0
    for l in range(NLAYERS):
        P = (jnp.dot(h, A1[l], preferred_element_type=jnp.float32)
             + bea[l])
        Q = jnp.dot(h, A2[l], preferred_element_type=jnp.float32)
        n2 = jnp.sum(Xn * Xn, axis=1, keepdims=True)
        G = jnp.dot(Xn, Xn.T, preferred_element_type=jnp.float32)
        D2 = n2 + n2.reshape(1, NPAD) - 2.0 * G
        Pb = P.astype(jnp.bfloat16)
        Qb = Q.astype(jnp.bfloat16)
        D2b = D2.astype(jnp.bfloat16)
        CEb = CEm.astype(jnp.bfloat16)
        E = (Pb.reshape(NPAD, 1, HIDDEN)
             + Qb.reshape(1, NPAD, HIDDEN)
             + D2b.reshape(NPAD, NPAD, 1) * wd3[l].reshape(1, 1, HIDDEN)
             + CEb.reshape(NPAD, NPAD, 1) * Dce[l].reshape(1, 1, HIDDEN))
        M1 = jnp.maximum(E, jnp.bfloat16(0.0)).reshape(NPAD * NPAD, HIDDEN)
        M2 = jnp.maximum(
            jnp.dot(M1, Web[l], preferred_element_type=jnp.float32)
            + beb[l], 0.0).astype(jnp.bfloat16)
        vcol = jnp.dot(M2, WxT[l],
                       preferred_element_type=jnp.float32)
        M2m = (M2.reshape(NPAD, NPAD, HIDDEN)
               * Mpair_bf.reshape(NPAD, NPAD, 1))
        xw_scr[...] = vcol.reshape(NPAD, NPAD)
        XW = jnp.tanh(xw_scr[...] + bx_s[l]) * Mpair
        agg = jnp.sum(M2m, axis=1).astype(jnp.float32) * invr
        S = jnp.sum(XW, axis=1, keepdims=True)
        Xn = Xn + (Xn * S
                   - jnp.dot(XW, Xn,
                             preferred_element_type=jnp.float32)) * invr
        nh = jnp.concatenate([h, agg], axis=1)
        h = (h
             + jnp.dot(
                 jnp.maximum(
                     jnp.dot(nh, Wna[l], preferred_element_type=jnp.float32)
                     + bna[l], 0.0),
                 Wnb[l], preferred_element_type=jnp.float32)
             + bnb[l])

    nH = jnp.dot(h, Wh2i[...], preferred_element_type=jnp.float32) + bh2i[...]
    exd = (Xn - Xn0) - eX
    pX = jnp.sum(jnp.sum(exd * exd, axis=1, keepdims=True) * gm)
    ehd = (nH - Hn) - eH
    pH = jnp.sum(jnp.sum(ehd * ehd, axis=1, keepdims=True) * gm)
    pD = jnp.sum(gm)
    lane = jax.lax.broadcasted_iota(jnp.int32, (1, HIDDEN), 1)
    vec = (jnp.where(lane == 0, pX, 0.0)
           + jnp.where(lane == 1, pH, 0.0)
           + jnp.where(lane == 2, pD, 0.0))
    out[...] = vec.reshape(1, 1, HIDDEN)


def kernel(H_0, X_0, cond_embedding, chain_ids, generate_mask, lengths, t,
           Win1, bin1, Win2, bin2, Win3, bin3, Wh2i, bh2i, Eemb,
           We_a, be_a, We_b, be_b, Wx, bx, Wn_a, bn_a, Wn_b, bn_b):
    kx, kh = jax.random.split(jax.random.key(42))
    eps_X = jax.random.normal(kx, X_0.shape, dtype=jnp.float32)
    eps_H = jax.random.normal(kh, H_0.shape, dtype=jnp.float32)
    betas = jnp.concatenate([jnp.zeros(1), jnp.linspace(1e-4, 0.02, NSTEPS)])
    abars = jnp.cumprod(1.0 - betas)
    ab_vec = abars[t].astype(jnp.float32)
    beta_vec = betas[t].astype(jnp.float32)

    src = jnp.asarray(_SRC_NP).reshape(-1)
    vf = jnp.asarray(_VALID_NP.astype(np.float32))

    H0p = H_0
    X0p = jnp.pad(X_0, ((0, 0), (0, XPAD - 3)))
    condp = cond_embedding
    eXp = jnp.pad(eps_X, ((0, 0), (0, XPAD - 3)))
    eHp = eps_H
    cidp = chain_ids.astype(jnp.int32)[src].reshape(NT, 1, NPAD)
    gmp = (generate_mask.astype(jnp.float32)[src].reshape(NT, 1, NPAD)
           * vf.reshape(NT, 1, NPAD))

    A1 = We_a[:, :HIDDEN, :]
    A2 = We_a[:, HIDDEN:2 * HIDDEN, :]
    wd3 = We_a[:, 2 * HIDDEN:2 * HIDDEN + 1, :]
    CeW = jnp.einsum('ec,lch->leh', Eemb, We_a[:, 2 * HIDDEN + 1:, :])
    Ce0 = CeW[:, 0:1, :].reshape(NLAYERS, 1, HIDDEN)
    Ce1 = CeW[:, 1:2, :].reshape(NLAYERS, 1, HIDDEN)
    Dce = Ce0 - Ce1
    bea2 = be_a.reshape(NLAYERS, 1, HIDDEN) + Ce1

    blk = lambda shp: pl.BlockSpec(shp, lambda b: (b, 0, 0))
    rep = lambda arr: pl.BlockSpec(arr.shape,
                                   lambda b, _n=arr.ndim: (0,) * _n)
    smem = pl.BlockSpec(memory_space=pltpu.SMEM)

    weights = [jnp.asarray(_FREQS_NP),
               Win1, bin1.reshape(1, HIDDEN), Win2, bin2.reshape(1, HIDDEN),
               Win3, bin3.reshape(1, HIDDEN),
               A1, A2, wd3.astype(jnp.bfloat16), Dce.astype(jnp.bfloat16),
               bea2,
               We_b.astype(jnp.bfloat16),
               be_b.reshape(NLAYERS, 1, HIDDEN),
               Wx.astype(jnp.bfloat16),
               Wn_a, bn_a.reshape(NLAYERS, 1, HIDDEN),
               Wn_b, bn_b.reshape(NLAYERS, 1, HIDDEN),
               Wh2i, bh2i.reshape(1, LATENT)]

    res = pl.pallas_call(
        _body,
        grid=(NT,),
        in_specs=[smem, smem, smem,
                  rep(H0p), rep(X0p), rep(condp), rep(eXp), rep(eHp),
                  blk((1, 1, NPAD)), blk((1, 1, NPAD))]
                 + [rep(w) for w in weights],
        out_specs=pl.BlockSpec((1, 1, HIDDEN), lambda b: (b, 0, 0)),
        out_shape=jax.ShapeDtypeStruct((NT, 1, HIDDEN), jnp.float32),
        scratch_shapes=[pltpu.VMEM((NPAD, NPAD), jnp.float32)],
        compiler_params=pltpu.CompilerParams(
            dimension_semantics=("parallel",)),
    )(ab_vec, beta_vec, bx.reshape(-1),
      H0p, X0p, condp, eXp, eHp, cidp, gmp, *weights)

    tot = jnp.sum(res, axis=(0, 1))
    denom = tot[2] + 1e-8
    return jnp.stack([tot[0] / denom, tot[1] / denom])


# speedup vs baseline: 22.9113x; 1.0892x over previous
import numpy as np
import jax
import jax.numpy as jnp
from jax.experimental import pallas as pl
from jax.experimental.pallas import tpu as pltpu

LATENT = 16
HIDDEN = 128
NSTEPS = 100
NLAYERS = 3
B = 64
N = 2016
NPAD = 64
XPAD = 8

NT = B // 2
_offs = (np.arange(B) * (np.arange(B) - 1)) // 2
_r = np.arange(NPAD)[None, :]
_p = np.arange(NT)[:, None]
_in1 = _r < _p
_in2 = (_r >= _p) & (_r < NPAD - 1)
_SRC_NP = np.where(
    _in1, _offs[np.minimum(_p, B - 1)] + _r,
    np.where(_in2, _offs[B - 1 - _p] + (_r - _p), 0)).astype(np.int32)
_VALID_NP = (_in1 | _in2)
_FREQS_NP = np.exp(-np.log(10000.0) * np.arange(HIDDEN // 2)
                   / (HIDDEN // 2 - 1)).astype(np.float32).reshape(1, -1)


def _body(ab_s, beta_s, bx_s,
          H0b, X0b, Cb, eXb, eHb, cidb, gmb,
          freqs_in, Win1, bin1, Win2, bin2, Win3, bin3,
          A1, A2, wd3, Dce, bea, Web, beb, WxT,
          Wna, bna, Wnb, bnb, Wh2i, bh2i,
          out, xw_scr):
    p = pl.program_id(0)

    iota_r = jax.lax.broadcasted_iota(jnp.int32, (NPAD, 1), 0)
    rb = iota_r < p
    rowsel = rb.astype(jnp.float32)
    ab1 = ab_s[p]
    ab2 = ab_s[B - 1 - p]
    abr = rowsel * ab1 + (1.0 - rowsel) * ab2
    sa = jnp.sqrt(abr)
    sb = jnp.sqrt(1.0 - abr)

    o1 = (p * (p - 1)) // 2
    q = B - 1 - p
    o2 = (q * (q - 1)) // 2 + q - NPAD

    def win(ref):
        w1 = ref[pl.ds(o1, NPAD), :]
        w2 = ref[pl.ds(o2, NPAD), :]
        w2s = jnp.concatenate([w2[1:NPAD], w2[0:1]], axis=0)
        return jnp.where(rb, w1, w2s)

    validr = (iota_r < NPAD - 1).astype(jnp.float32)
    gm = win(gmb) * validr
    H0 = win(H0b)
    X0 = jnp.pad(win(X0b), ((0, 0), (0, XPAD - 3)))
    cond = win(Cb)
    eX = jnp.pad(win(eXb), ((0, 0), (0, XPAD - 3)))
    eH = win(eHb)

    Xn0 = gm * (sa * X0 + sb * eX) + (1.0 - gm) * X0
    Hn = gm * (sa * H0 + sb * eH) + (1.0 - gm) * H0

    freqs = freqs_in[...]
    te1 = beta_s[p] * freqs
    te2 = beta_s[B - 1 - p] * freqs
    te1 = jnp.concatenate([jnp.sin(te1), jnp.cos(te1)], axis=1)
    te2 = jnp.concatenate([jnp.sin(te2), jnp.cos(te2)], axis=1)
    te = rowsel * te1 + (1.0 - rowsel) * te2

    feat = jnp.concatenate([Hn, cond, te], axis=1)
    h = jnp.maximum(
        jnp.dot(feat, Win1[...], preferred_element_type=jnp.float32)
        + bin1[...], 0.0)
    h = jnp.maximum(
        jnp.dot(h, Win2[...], preferred_element_type=jnp.float32)
        + bin2[...], 0.0)
    h = jnp.dot(h, Win3[...], preferred_element_type=jnp.float32) + bin3[...]

    cidw = win(cidb)
    CEm = (cidw == cidw.reshape(1, NPAD)).astype(jnp.float32)
    jota = jax.lax.broadcasted_iota(jnp.int32, (1, NPAD), 1)
    validj = (jota < NPAD - 1).astype(jnp.float32)
    rsj = rowsel.reshape(1, NPAD)
    Mpair = (rowsel * rsj + (1.0 - rowsel) * (1.0 - rsj)) * validj
    pf = p.astype(jnp.float32)
    invr = rowsel / (pf + 1.0) + (1.0 - rowsel) / (64.0 - pf)

    Mpair_bf = Mpair.astype(jnp.bfloat16)
    Xn = Xn0
    for l in range(NLAYERS):
        P = (jnp.dot(h, A1[l], preferred_element_type=jnp.float32)
             + bea[l])
        Q = jnp.dot(h, A2[l], preferred_element_type=jnp.float32)
        n2 = jnp.sum(Xn * Xn, axis=1, keepdims=True)
        G = jnp.dot(Xn, Xn.T, preferred_element_type=jnp.float32)
        D2 = n2 + n2.reshape(1, NPAD) - 2.0 * G
        Pb = P.astype(jnp.bfloat16)
        Qb = Q.astype(jnp.bfloat16)
        D2b = D2.astype(jnp.bfloat16)
        CEb = CEm.astype(jnp.bfloat16)
        E = (Pb.reshape(NPAD, 1, HIDDEN)
             + Qb.reshape(1, NPAD, HIDDEN)
             + D2b.reshape(NPAD, NPAD, 1) * wd3[l].reshape(1, 1, HIDDEN)
             + CEb.reshape(NPAD, NPAD, 1) * Dce[l].reshape(1, 1, HIDDEN))
        M1 = jnp.maximum(E, jnp.bfloat16(0.0)).reshape(NPAD * NPAD, HIDDEN)
        M2 = jnp.maximum(
            jnp.dot(M1, Web[l], preferred_element_type=jnp.float32)
            + beb[l], 0.0).astype(jnp.bfloat16)
        vcol = jnp.dot(M2, WxT[l],
                       preferred_element_type=jnp.float32)
        M2m = (M2.reshape(NPAD, NPAD, HIDDEN)
               * Mpair_bf.reshape(NPAD, NPAD, 1))
        xw_scr[...] = vcol.reshape(NPAD, NPAD)
        XW = jnp.tanh(xw_scr[...] + bx_s[l]) * Mpair
        agg = jnp.sum(M2m, axis=1).astype(jnp.float32) * invr
        S = jnp.sum(XW, axis=1, keepdims=True)
        Xn = Xn + (Xn * S
                   - jnp.dot(XW, Xn,
                             preferred_element_type=jnp.float32)) * invr
        nh = jnp.concatenate([h, agg], axis=1)
        h = (h
             + jnp.dot(
                 jnp.maximum(
                     jnp.dot(nh, Wna[l], preferred_element_type=jnp.float32)
                     + bna[l], 0.0),
                 Wnb[l], preferred_element_type=jnp.float32)
             + bnb[l])

    nH = jnp.dot(h, Wh2i[...], preferred_element_type=jnp.float32) + bh2i[...]
    exd = (Xn - Xn0) - eX
    pX = jnp.sum(jnp.sum(exd * exd, axis=1, keepdims=True) * gm)
    ehd = (nH - Hn) - eH
    pH = jnp.sum(jnp.sum(ehd * ehd, axis=1, keepdims=True) * gm)
    pD = jnp.sum(gm)
    lane = jax.lax.broadcasted_iota(jnp.int32, (1, HIDDEN), 1)
    vec = (jnp.where(lane == 0, pX, 0.0)
           + jnp.where(lane == 1, pH, 0.0)
           + jnp.where(lane == 2, pD, 0.0))
    out[...] = vec.reshape(1, 1, HIDDEN)


def kernel(H_0, X_0, cond_embedding, chain_ids, generate_mask, lengths, t,
           Win1, bin1, Win2, bin2, Win3, bin3, Wh2i, bh2i, Eemb,
           We_a, be_a, We_b, be_b, Wx, bx, Wn_a, bn_a, Wn_b, bn_b):
    kx, kh = jax.random.split(jax.random.key(42))
    eps_X = jax.random.normal(kx, X_0.shape, dtype=jnp.float32)
    eps_H = jax.random.normal(kh, H_0.shape, dtype=jnp.float32)
    betas = jnp.concatenate([jnp.zeros(1), jnp.linspace(1e-4, 0.02, NSTEPS)])
    abars = jnp.cumprod(1.0 - betas)
    ab_vec = abars[t].astype(jnp.float32)
    beta_vec = betas[t].astype(jnp.float32)

    H0p = H_0
    X0p = X_0
    condp = cond_embedding
    eXp = eps_X
    eHp = eps_H
    cidp = chain_ids.astype(jnp.int32).reshape(N, 1)
    gmp = generate_mask.astype(jnp.float32).reshape(N, 1)

    A1 = We_a[:, :HIDDEN, :]
    A2 = We_a[:, HIDDEN:2 * HIDDEN, :]
    wd3 = We_a[:, 2 * HIDDEN:2 * HIDDEN + 1, :]
    CeW = jnp.einsum('ec,lch->leh', Eemb, We_a[:, 2 * HIDDEN + 1:, :])
    Ce0 = CeW[:, 0:1, :].reshape(NLAYERS, 1, HIDDEN)
    Ce1 = CeW[:, 1:2, :].reshape(NLAYERS, 1, HIDDEN)
    Dce = Ce0 - Ce1
    bea2 = be_a.reshape(NLAYERS, 1, HIDDEN) + Ce1

    blk = lambda shp: pl.BlockSpec(shp, lambda b: (b, 0, 0))
    rep = lambda arr: pl.BlockSpec(arr.shape,
                                   lambda b, _n=arr.ndim: (0,) * _n)
    smem = pl.BlockSpec(memory_space=pltpu.SMEM)

    weights = [jnp.asarray(_FREQS_NP),
               Win1, bin1.reshape(1, HIDDEN), Win2, bin2.reshape(1, HIDDEN),
               Win3, bin3.reshape(1, HIDDEN),
               A1, A2, wd3.astype(jnp.bfloat16), Dce.astype(jnp.bfloat16),
               bea2,
               We_b.astype(jnp.bfloat16),
               be_b.reshape(NLAYERS, 1, HIDDEN),
               Wx.astype(jnp.bfloat16),
               Wn_a, bn_a.reshape(NLAYERS, 1, HIDDEN),
               Wn_b, bn_b.reshape(NLAYERS, 1, HIDDEN),
               Wh2i, bh2i.reshape(1, LATENT)]

    res = pl.pallas_call(
        _body,
        grid=(NT,),
        in_specs=[smem, smem, smem,
                  rep(H0p), rep(X0p), rep(condp), rep(eXp), rep(eHp),
                  rep(cidp), rep(gmp)]
                 + [rep(w) for w in weights],
        out_specs=pl.BlockSpec((1, 1, HIDDEN), lambda b: (b, 0, 0)),
        out_shape=jax.ShapeDtypeStruct((NT, 1, HIDDEN), jnp.float32),
        scratch_shapes=[pltpu.VMEM((NPAD, NPAD), jnp.float32)],
        compiler_params=pltpu.CompilerParams(
            dimension_semantics=("parallel",)),
    )(ab_vec, beta_vec, bx.reshape(-1),
      H0p, X0p, condp, eXp, eHp, cidp, gmp, *weights)

    tot = jnp.sum(res, axis=(0, 1))
    denom = tot[2] + 1e-8
    return jnp.stack([tot[0] / denom, tot[1] / denom])


# speedup vs baseline: 23.4080x; 1.0217x over previous
import numpy as np
import jax
import jax.numpy as jnp
from jax.experimental import pallas as pl
from jax.experimental.pallas import tpu as pltpu

LATENT = 16
HIDDEN = 128
NSTEPS = 100
NLAYERS = 3
B = 64
N = 2016
NPAD = 64
XPAD = 8

NT = B // 2
_offs = (np.arange(B) * (np.arange(B) - 1)) // 2
_r = np.arange(NPAD)[None, :]
_p = np.arange(NT)[:, None]
_in1 = _r < _p
_in2 = (_r >= _p) & (_r < NPAD - 1)
_SRC_NP = np.where(
    _in1, _offs[np.minimum(_p, B - 1)] + _r,
    np.where(_in2, _offs[B - 1 - _p] + (_r - _p), 0)).astype(np.int32)
_VALID_NP = (_in1 | _in2)
_FREQS_NP = np.exp(-np.log(10000.0) * np.arange(HIDDEN // 2)
                   / (HIDDEN // 2 - 1)).astype(np.float32).reshape(1, -1)


def _body(ab_s, beta_s, bx_s,
          H0b, X0b, Cb, eXb, eHb, cidb, gmb,
          freqs_in, Win1, bin1, Win2, bin2, Win3, bin3,
          Wea, bea, Eemb_r, Web, beb, Wx_r,
          Wna, bna, Wnb, bnb, Wh2i, bh2i,
          out, xw_scr):
    p = pl.program_id(0)

    iota_r = jax.lax.broadcasted_iota(jnp.int32, (NPAD, 1), 0)
    rb = iota_r < p
    rowsel = rb.astype(jnp.float32)
    ab1 = ab_s[p]
    ab2 = ab_s[B - 1 - p]
    abr = rowsel * ab1 + (1.0 - rowsel) * ab2
    sa = jnp.sqrt(abr)
    sb = jnp.sqrt(1.0 - abr)

    o1 = (p * (p - 1)) // 2
    q = B - 1 - p
    o2 = (q * (q - 1)) // 2 + q - NPAD

    def win(ref):
        w1 = ref[pl.ds(o1, NPAD), :]
        w2 = ref[pl.ds(o2, NPAD), :]
        w2s = jnp.concatenate([w2[1:NPAD], w2[0:1]], axis=0)
        return jnp.where(rb, w1, w2s)

    validr = (iota_r < NPAD - 1).astype(jnp.float32)
    gm = win(gmb) * validr
    H0 = win(H0b)
    X0 = jnp.pad(win(X0b), ((0, 0), (0, XPAD - 3)))
    cond = win(Cb)
    eX = jnp.pad(win(eXb), ((0, 0), (0, XPAD - 3)))
    eH = win(eHb)

    Xn0 = gm * (sa * X0 + sb * eX) + (1.0 - gm) * X0
    Hn = gm * (sa * H0 + sb * eH) + (1.0 - gm) * H0

    freqs = freqs_in[...]
    te1 = beta_s[p] * freqs
    te2 = beta_s[B - 1 - p] * freqs
    te1 = jnp.concatenate([jnp.sin(te1), jnp.cos(te1)], axis=1)
    te2 = jnp.concatenate([jnp.sin(te2), jnp.cos(te2)], axis=1)
    te = rowsel * te1 + (1.0 - rowsel) * te2

    feat = jnp.concatenate([Hn, cond, te], axis=1)
    h = jnp.maximum(
        jnp.dot(feat, Win1[...], preferred_element_type=jnp.float32)
        + bin1[...], 0.0)
    h = jnp.maximum(
        jnp.dot(h, Win2[...], preferred_element_type=jnp.float32)
        + bin2[...], 0.0)
    h = jnp.dot(h, Win3[...], preferred_element_type=jnp.float32) + bin3[...]

    cidw = win(cidb)
    CEm = (cidw == cidw.reshape(1, NPAD)).astype(jnp.float32)
    jota = jax.lax.broadcasted_iota(jnp.int32, (1, NPAD), 1)
    validj = (jota < NPAD - 1).astype(jnp.float32)
    rsj = rowsel.reshape(1, NPAD)
    Mpair = (rowsel * rsj + (1.0 - rowsel) * (1.0 - rsj)) * validj
    pf = p.astype(jnp.float32)
    invr = rowsel / (pf + 1.0) + (1.0 - rowsel) / (64.0 - pf)

    Mpair_bf = Mpair.astype(jnp.bfloat16)
    Xn = Xn0
    for l in range(NLAYERS):
        CeW = jnp.dot(Eemb_r[...], Wea[l, 2 * HIDDEN + 1:, :],
                      preferred_element_type=jnp.float32)
        ce1 = CeW[1:2, :]
        dce = (CeW[0:1, :] - ce1).astype(jnp.bfloat16)
        wdl = Wea[l, 2 * HIDDEN:2 * HIDDEN + 1, :].astype(jnp.bfloat16)
        P = (jnp.dot(h, Wea[l, :HIDDEN, :],
                     preferred_element_type=jnp.float32)
             + bea[l] + ce1)
        Q = jnp.dot(h, Wea[l, HIDDEN:2 * HIDDEN, :],
                    preferred_element_type=jnp.float32)
        n2 = jnp.sum(Xn * Xn, axis=1, keepdims=True)
        G = jnp.dot(Xn, Xn.T, preferred_element_type=jnp.float32)
        D2 = n2 + n2.reshape(1, NPAD) - 2.0 * G
        Pb = P.astype(jnp.bfloat16)
        Qb = Q.astype(jnp.bfloat16)
        D2b = D2.astype(jnp.bfloat16)
        CEb = CEm.astype(jnp.bfloat16)
        E = (Pb.reshape(NPAD, 1, HIDDEN)
             + Qb.reshape(1, NPAD, HIDDEN)
             + D2b.reshape(NPAD, NPAD, 1) * wdl.reshape(1, 1, HIDDEN)
             + CEb.reshape(NPAD, NPAD, 1) * dce.reshape(1, 1, HIDDEN))
        M1 = jnp.maximum(E, jnp.bfloat16(0.0)).reshape(NPAD * NPAD, HIDDEN)
        M2 = jnp.maximum(
            jnp.dot(M1, Web[l].astype(jnp.bfloat16),
                    preferred_element_type=jnp.float32)
            + beb[l], 0.0).astype(jnp.bfloat16)
        vcol = jnp.dot(M2, Wx_r[l].astype(jnp.bfloat16),
                       preferred_element_type=jnp.float32)
        M2m = (M2.reshape(NPAD, NPAD, HIDDEN)
               * Mpair_bf.reshape(NPAD, NPAD, 1))
        xw_scr[...] = vcol.reshape(NPAD, NPAD)
        XW = jnp.tanh(xw_scr[...] + bx_s[l]) * Mpair
        agg = jnp.sum(M2m, axis=1).astype(jnp.float32) * invr
        S = jnp.sum(XW, axis=1, keepdims=True)
        Xn = Xn + (Xn * S
                   - jnp.dot(XW, Xn,
                             preferred_element_type=jnp.float32)) * invr
        nh = jnp.concatenate([h, agg], axis=1)
        h = (h
             + jnp.dot(
                 jnp.maximum(
                     jnp.dot(nh, Wna[l], preferred_element_type=jnp.float32)
                     + bna[l], 0.0),
                 Wnb[l], preferred_element_type=jnp.float32)
             + bnb[l])

    nH = jnp.dot(h, Wh2i[...], preferred_element_type=jnp.float32) + bh2i[...]
    exd = (Xn - Xn0) - eX
    pX = jnp.sum(jnp.sum(exd * exd, axis=1, keepdims=True) * gm)
    ehd = (nH - Hn) - eH
    pH = jnp.sum(jnp.sum(ehd * ehd, axis=1, keepdims=True) * gm)
    pD = jnp.sum(gm)
    lane = jax.lax.broadcasted_iota(jnp.int32, (1, HIDDEN), 1)
    vec = (jnp.where(lane == 0, pX, 0.0)
           + jnp.where(lane == 1, pH, 0.0)
           + jnp.where(lane == 2, pD, 0.0))
    out[...] = vec.reshape(1, 1, HIDDEN)


def kernel(H_0, X_0, cond_embedding, chain_ids, generate_mask, lengths, t,
           Win1, bin1, Win2, bin2, Win3, bin3, Wh2i, bh2i, Eemb,
           We_a, be_a, We_b, be_b, Wx, bx, Wn_a, bn_a, Wn_b, bn_b):
    kx, kh = jax.random.split(jax.random.key(42))
    eps_X = jax.random.normal(kx, X_0.shape, dtype=jnp.float32)
    eps_H = jax.random.normal(kh, H_0.shape, dtype=jnp.float32)
    betas = jnp.concatenate([jnp.zeros(1), jnp.linspace(1e-4, 0.02, NSTEPS)])
    abars = jnp.cumprod(1.0 - betas)
    ab_vec = abars[t].astype(jnp.float32)
    beta_vec = betas[t].astype(jnp.float32)

    H0p = H_0
    X0p = X_0
    condp = cond_embedding
    eXp = eps_X
    eHp = eps_H
    cidp = chain_ids.astype(jnp.int32).reshape(N, 1)
    gmp = generate_mask.astype(jnp.float32).reshape(N, 1)

    rep = lambda arr: pl.BlockSpec(arr.shape,
                                   lambda b, _n=arr.ndim: (0,) * _n)
    smem = pl.BlockSpec(memory_space=pltpu.SMEM)

    weights = [jnp.asarray(_FREQS_NP),
               Win1, bin1.reshape(1, HIDDEN), Win2, bin2.reshape(1, HIDDEN),
               Win3, bin3.reshape(1, HIDDEN),
               We_a, be_a.reshape(NLAYERS, 1, HIDDEN), Eemb,
               We_b, be_b.reshape(NLAYERS, 1, HIDDEN), Wx,
               Wn_a, bn_a.reshape(NLAYERS, 1, HIDDEN),
               Wn_b, bn_b.reshape(NLAYERS, 1, HIDDEN),
               Wh2i, bh2i.reshape(1, LATENT)]

    res = pl.pallas_call(
        _body,
        grid=(NT,),
        in_specs=[smem, smem, smem,
                  rep(H0p), rep(X0p), rep(condp), rep(eXp), rep(eHp),
                  rep(cidp), rep(gmp)]
                 + [rep(w) for w in weights],
        out_specs=pl.BlockSpec((1, 1, HIDDEN), lambda b: (b, 0, 0)),
        out_shape=jax.ShapeDtypeStruct((NT, 1, HIDDEN), jnp.float32),
        scratch_shapes=[pltpu.VMEM((NPAD, NPAD), jnp.float32)],
        compiler_params=pltpu.CompilerParams(
            dimension_semantics=("arbitrary",)),
    )(ab_vec, beta_vec, bx.reshape(-1),
      H0p, X0p, condp, eXp, eHp, cidp, gmp, *weights)

    tot = jnp.sum(res, axis=(0, 1))
    denom = tot[2] + 1e-8
    return jnp.stack([tot[0] / denom, tot[1] / denom])


# speedup vs baseline: 23.7524x; 1.0147x over previous
import numpy as np
import jax
import jax.numpy as jnp
from jax.experimental import pallas as pl
from jax.experimental.pallas import tpu as pltpu

LATENT = 16
HIDDEN = 128
NSTEPS = 100
NLAYERS = 3
B = 64
N = 2016
NPAD = 64
XPAD = 8

NT = B // 2
_offs = (np.arange(B) * (np.arange(B) - 1)) // 2
_r = np.arange(NPAD)[None, :]
_p = np.arange(NT)[:, None]
_in1 = _r < _p
_in2 = (_r >= _p) & (_r < NPAD - 1)
_SRC_NP = np.where(
    _in1, _offs[np.minimum(_p, B - 1)] + _r,
    np.where(_in2, _offs[B - 1 - _p] + (_r - _p), 0)).astype(np.int32)
_VALID_NP = (_in1 | _in2)
_FREQS_NP = np.exp(-np.log(10000.0) * np.arange(HIDDEN // 2)
                   / (HIDDEN // 2 - 1)).astype(np.float32).reshape(1, -1)


def _body(t_s, ab_tab, beta_tab, bx_s,
          H0b, X0b, Cb, eXb, eHb, cidb, gmb,
          freqs_in, Win1, bin1, Win2, bin2, Win3, bin3,
          Wea, bea, Eemb_r, Web, beb, Wx_r,
          Wna, bna, Wnb, bnb, Wh2i, bh2i,
          out, xw_scr):
    p = pl.program_id(0)

    iota_r = jax.lax.broadcasted_iota(jnp.int32, (NPAD, 1), 0)
    rb = iota_r < p
    rowsel = rb.astype(jnp.float32)
    t1 = t_s[p]
    t2 = t_s[B - 1 - p]
    ab1 = ab_tab[t1]
    ab2 = ab_tab[t2]
    abr = rowsel * ab1 + (1.0 - rowsel) * ab2
    sa = jnp.sqrt(abr)
    sb = jnp.sqrt(1.0 - abr)

    o1 = (p * (p - 1)) // 2
    q = B - 1 - p
    o2 = (q * (q - 1)) // 2 + q - NPAD

    def win(ref):
        w1 = ref[pl.ds(o1, NPAD), :]
        w2 = ref[pl.ds(o2, NPAD), :]
        w2s = jnp.concatenate([w2[1:NPAD], w2[0:1]], axis=0)
        return jnp.where(rb, w1, w2s)

    validr = (iota_r < NPAD - 1).astype(jnp.float32)
    gm = win(gmb) * validr
    H0 = win(H0b)
    X0 = jnp.pad(win(X0b), ((0, 0), (0, XPAD - 3)))
    cond = win(Cb)
    eX = jnp.pad(win(eXb), ((0, 0), (0, XPAD - 3)))
    eH = win(eHb)

    Xn0 = gm * (sa * X0 + sb * eX) + (1.0 - gm) * X0
    Hn = gm * (sa * H0 + sb * eH) + (1.0 - gm) * H0

    freqs = freqs_in[...]
    te1 = beta_tab[t1] * freqs
    te2 = beta_tab[t2] * freqs
    te1 = jnp.concatenate([jnp.sin(te1), jnp.cos(te1)], axis=1)
    te2 = jnp.concatenate([jnp.sin(te2), jnp.cos(te2)], axis=1)
    te = rowsel * te1 + (1.0 - rowsel) * te2

    feat = jnp.concatenate([Hn, cond, te], axis=1)
    h = jnp.maximum(
        jnp.dot(feat, Win1[...], preferred_element_type=jnp.float32)
        + bin1[...], 0.0)
    h = jnp.maximum(
        jnp.dot(h, Win2[...], preferred_element_type=jnp.float32)
        + bin2[...], 0.0)
    h = jnp.dot(h, Win3[...], preferred_element_type=jnp.float32) + bin3[...]

    cidw = win(cidb)
    CEm = (cidw == cidw.reshape(1, NPAD)).astype(jnp.float32)
    jota = jax.lax.broadcasted_iota(jnp.int32, (1, NPAD), 1)
    validj = (jota < NPAD - 1).astype(jnp.float32)
    rsj = rowsel.reshape(1, NPAD)
    Mpair = (rowsel * rsj + (1.0 - rowsel) * (1.0 - rsj)) * validj
    pf = p.astype(jnp.float32)
    invr = rowsel / (pf + 1.0) + (1.0 - rowsel) / (64.0 - pf)

    Mpair_bf = Mpair.astype(jnp.bfloat16)
    Xn = Xn0
    for l in range(NLAYERS):
        CeW = jnp.dot(Eemb_r[...], Wea[l, 2 * HIDDEN + 1:, :],
                      preferred_element_type=jnp.float32)
        ce1 = CeW[1:2, :]
        dce = (CeW[0:1, :] - ce1).astype(jnp.bfloat16)
        wdl = Wea[l, 2 * HIDDEN:2 * HIDDEN + 1, :].astype(jnp.bfloat16)
        P = (jnp.dot(h, Wea[l, :HIDDEN, :],
                     preferred_element_type=jnp.float32)
             + bea[l] + ce1)
        Q = jnp.dot(h, Wea[l, HIDDEN:2 * HIDDEN, :],
                    preferred_element_type=jnp.float32)
        n2 = jnp.sum(Xn * Xn, axis=1, keepdims=True)
        G = jnp.dot(Xn, Xn.T, preferred_element_type=jnp.float32)
        D2 = n2 + n2.reshape(1, NPAD) - 2.0 * G
        Pb = P.astype(jnp.bfloat16)
        Qb = Q.astype(jnp.bfloat16)
        D2b = D2.astype(jnp.bfloat16)
        CEb = CEm.astype(jnp.bfloat16)
        E = (Pb.reshape(NPAD, 1, HIDDEN)
             + Qb.reshape(1, NPAD, HIDDEN)
             + D2b.reshape(NPAD, NPAD, 1) * wdl.reshape(1, 1, HIDDEN)
             + CEb.reshape(NPAD, NPAD, 1) * dce.reshape(1, 1, HIDDEN))
        M1 = jnp.maximum(E, jnp.bfloat16(0.0)).reshape(NPAD * NPAD, HIDDEN)
        M2 = jnp.maximum(
            jnp.dot(M1, Web[l].astype(jnp.bfloat16),
                    preferred_element_type=jnp.float32)
            + beb[l], 0.0).astype(jnp.bfloat16)
        vcol = jnp.dot(M2, Wx_r[l].astype(jnp.bfloat16),
                       preferred_element_type=jnp.float32)
        M2m = (M2.reshape(NPAD, NPAD, HIDDEN)
               * Mpair_bf.reshape(NPAD, NPAD, 1))
        xw_scr[...] = vcol.reshape(NPAD, NPAD)
        XW = jnp.tanh(xw_scr[...] + bx_s[l]) * Mpair
        agg = jnp.sum(M2m, axis=1).astype(jnp.float32) * invr
        S = jnp.sum(XW, axis=1, keepdims=True)
        Xn = Xn + (Xn * S
                   - jnp.dot(XW, Xn,
                             preferred_element_type=jnp.float32)) * invr
        nh = jnp.concatenate([h, agg], axis=1)
        h = (h
             + jnp.dot(
                 jnp.maximum(
                     jnp.dot(nh, Wna[l], preferred_element_type=jnp.float32)
                     + bna[l], 0.0),
                 Wnb[l], preferred_element_type=jnp.float32)
             + bnb[l])

    nH = jnp.dot(h, Wh2i[...], preferred_element_type=jnp.float32) + bh2i[...]
    exd = (Xn - Xn0) - eX
    pX = jnp.sum(jnp.sum(exd * exd, axis=1, keepdims=True) * gm)
    ehd = (nH - Hn) - eH
    pH = jnp.sum(jnp.sum(ehd * ehd, axis=1, keepdims=True) * gm)
    pD = jnp.sum(gm)

    @pl.when(p == 0)
    def _init():
        out[0] = 0.0
        out[1] = 0.0
        out[2] = 0.0

    out[0] += pX
    out[1] += pH
    out[2] += pD


def kernel(H_0, X_0, cond_embedding, chain_ids, generate_mask, lengths, t,
           Win1, bin1, Win2, bin2, Win3, bin3, Wh2i, bh2i, Eemb,
           We_a, be_a, We_b, be_b, Wx, bx, Wn_a, bn_a, Wn_b, bn_b):
    kx, kh = jax.random.split(jax.random.key(42))
    eps_X = jax.random.normal(kx, X_0.shape, dtype=jnp.float32)
    eps_H = jax.random.normal(kh, H_0.shape, dtype=jnp.float32)
    betas = jnp.concatenate([jnp.zeros(1), jnp.linspace(1e-4, 0.02, NSTEPS)])
    abars = jnp.cumprod(1.0 - betas)

    H0p = H_0
    X0p = X_0
    condp = cond_embedding
    eXp = eps_X
    eHp = eps_H
    cidp = chain_ids.astype(jnp.int32).reshape(N, 1)
    gmp = generate_mask.astype(jnp.float32).reshape(N, 1)

    rep = lambda arr: pl.BlockSpec(arr.shape,
                                   lambda b, _n=arr.ndim: (0,) * _n)
    smem = pl.BlockSpec(memory_space=pltpu.SMEM)

    weights = [jnp.asarray(_FREQS_NP),
               Win1, bin1.reshape(1, HIDDEN), Win2, bin2.reshape(1, HIDDEN),
               Win3, bin3.reshape(1, HIDDEN),
               We_a, be_a.reshape(NLAYERS, 1, HIDDEN), Eemb,
               We_b, be_b.reshape(NLAYERS, 1, HIDDEN), Wx,
               Wn_a, bn_a.reshape(NLAYERS, 1, HIDDEN),
               Wn_b, bn_b.reshape(NLAYERS, 1, HIDDEN),
               Wh2i, bh2i.reshape(1, LATENT)]

    res = pl.pallas_call(
        _body,
        grid=(NT,),
        in_specs=[smem, smem, smem, smem,
                  rep(H0p), rep(X0p), rep(condp), rep(eXp), rep(eHp),
                  rep(cidp), rep(gmp)]
                 + [rep(w) for w in weights],
        out_specs=pl.BlockSpec(memory_space=pltpu.SMEM),
        out_shape=jax.ShapeDtypeStruct((3,), jnp.float32),
        scratch_shapes=[pltpu.VMEM((NPAD, NPAD), jnp.float32)],
        compiler_params=pltpu.CompilerParams(
            dimension_semantics=("arbitrary",)),
    )(t.astype(jnp.int32), abars.astype(jnp.float32),
      betas.astype(jnp.float32), bx.reshape(-1),
      H0p, X0p, condp, eXp, eHp, cidp, gmp, *weights)

    denom = res[2] + 1e-8
    return jnp.stack([res[0] / denom, res[1] / denom])
